# Initial kernel scaffold; baseline (speedup 1.0000x reference)
import functools

import jax
import jax.numpy as jnp
from jax import lax
from jax.experimental import pallas as pl
from jax.experimental.pallas import tpu as pltpu
from jax.experimental.pallas import tpu_sc as plsc

N = 10000
D = 128
E = 320000
OUT = 2

NC = 2
NS = 16
NW = NC * NS
EPW = E // NW
CH = 80
NCH = EPW // CH

BLK = 2000
GRID = N // BLK


def _sc_mesh():
    return plsc.VectorSubcoreMesh(core_axis_name="c", subcore_axis_name="s")


def _deg_body(dst_hbm, zeros_hbm, out_hbm, idx_v, ones_v, acc_s):
    cid = lax.axis_index("c")
    sid = lax.axis_index("s")
    wid = cid * NS + sid
    for j in range(CH // 16):
        ones_v[pl.ds(j * 16, 16)] = jnp.ones((16,), jnp.float32)

    @pl.when(sid == 0)
    def _():
        pltpu.sync_copy(zeros_hbm, acc_s)

    plsc.subcore_barrier()
    pltpu.sync_copy(dst_hbm.at[wid], idx_v)

    def body(j, carry):
        pltpu.sync_copy(ones_v, acc_s.at[idx_v.at[j]], add=True)
        return carry

    lax.fori_loop(0, NCH, body, 0)
    plsc.subcore_barrier()

    @pl.when(sid == 0)
    def _():
        pltpu.sync_copy(acc_s, out_hbm.at[cid])


_deg_kernel = functools.partial(
    pl.kernel,
    mesh=_sc_mesh(),
    out_type=jax.ShapeDtypeStruct((NC, N), jnp.float32),
    scratch_types=[
        pltpu.VMEM((NCH, CH), jnp.int32),
        pltpu.VMEM((CH,), jnp.float32),
        pltpu.VMEM_SHARED((N,), jnp.float32),
    ],
)(_deg_body)


def _agg_body(src_hbm, dst_hbm, g_hbm, zeros_hbm, out_hbm,
              sidx_v, didx_v, rows_v, acc_s, sem):
    cid = lax.axis_index("c")
    sid = lax.axis_index("s")
    wid = cid * NS + sid
    rows_per = N // NS

    pltpu.sync_copy(zeros_hbm.at[pl.ds(sid * rows_per, rows_per)],
                    acc_s.at[pl.ds(sid * rows_per, rows_per)])
    plsc.subcore_barrier()

    pltpu.sync_copy(src_hbm.at[wid], sidx_v)
    pltpu.sync_copy(dst_hbm.at[wid], didx_v)

    def body(j, carry):
        pltpu.async_copy(g_hbm.at[sidx_v.at[j]], rows_v, sem).wait()
        pltpu.sync_copy(rows_v, acc_s.at[didx_v.at[j]], add=True)
        return carry

    lax.fori_loop(0, NCH, body, 0)
    plsc.subcore_barrier()

    pltpu.sync_copy(acc_s.at[pl.ds(sid * rows_per, rows_per)],
                    out_hbm.at[cid, pl.ds(sid * rows_per, rows_per)])


_agg_kernel = functools.partial(
    pl.kernel,
    mesh=_sc_mesh(),
    out_type=jax.ShapeDtypeStruct((NC, N, D), jnp.float32),
    scratch_types=[
        pltpu.VMEM((NCH, CH), jnp.int32),
        pltpu.VMEM((NCH, CH), jnp.int32),
        pltpu.VMEM((CH, D), jnp.float32),
        pltpu.VMEM_SHARED((N, D), jnp.float32),
        pltpu.SemaphoreType.DMA,
    ],
)(_agg_body)


def _mm1_body(x_ref, d0_ref, d1_ref, w1_ref, g_ref, dinv_ref):
    deg = d0_ref[...] + d1_ref[...] + 1.0
    dinv = lax.rsqrt(deg)
    h = jnp.dot(x_ref[...], w1_ref[...], preferred_element_type=jnp.float32)
    g_ref[...] = h * dinv
    dinv_ref[...] = dinv


def _mm1(x, d0, d1, W1):
    return pl.pallas_call(
        _mm1_body,
        grid=(GRID,),
        in_specs=[
            pl.BlockSpec((BLK, D), lambda i: (i, 0)),
            pl.BlockSpec((BLK, 1), lambda i: (i, 0)),
            pl.BlockSpec((BLK, 1), lambda i: (i, 0)),
            pl.BlockSpec((D, D), lambda i: (0, 0)),
        ],
        out_specs=[
            pl.BlockSpec((BLK, D), lambda i: (i, 0)),
            pl.BlockSpec((BLK, 1), lambda i: (i, 0)),
        ],
        out_shape=[
            jax.ShapeDtypeStruct((N, D), jnp.float32),
            jax.ShapeDtypeStruct((N, 1), jnp.float32),
        ],
    )(x, d0, d1, W1)


def _mlp_body(p0_ref, p1_ref, g_ref, dinv_ref, b1_ref, w2_ref, b2_ref,
              w3_ref, b3_ref, wl_ref, bl_ref, out_ref):
    conv = (p0_ref[...] + p1_ref[...] + g_ref[...]) * dinv_ref[...] + b1_ref[...]
    h = jnp.maximum(conv, 0.0)
    h = jnp.dot(h, w2_ref[...], preferred_element_type=jnp.float32) + b2_ref[...]
    h = jnp.maximum(h, 0.0)
    h = jnp.dot(h, w3_ref[...], preferred_element_type=jnp.float32) + b3_ref[...]
    h = jnp.maximum(h, 0.0)
    out_ref[...] = (
        jnp.dot(h, wl_ref[...], preferred_element_type=jnp.float32) + bl_ref[...]
    )


def _mlp(p0, p1, g, dinv, b1, W2, b2, W3, b3, Wl, bl):
    return pl.pallas_call(
        _mlp_body,
        grid=(GRID,),
        in_specs=[
            pl.BlockSpec((BLK, D), lambda i: (i, 0)),
            pl.BlockSpec((BLK, D), lambda i: (i, 0)),
            pl.BlockSpec((BLK, D), lambda i: (i, 0)),
            pl.BlockSpec((BLK, 1), lambda i: (i, 0)),
            pl.BlockSpec((1, D), lambda i: (0, 0)),
            pl.BlockSpec((D, D), lambda i: (0, 0)),
            pl.BlockSpec((1, D), lambda i: (0, 0)),
            pl.BlockSpec((D, D), lambda i: (0, 0)),
            pl.BlockSpec((1, D), lambda i: (0, 0)),
            pl.BlockSpec((D, OUT), lambda i: (0, 0)),
            pl.BlockSpec((1, OUT), lambda i: (0, 0)),
        ],
        out_specs=pl.BlockSpec((BLK, OUT), lambda i: (i, 0)),
        out_shape=jax.ShapeDtypeStruct((N, OUT), jnp.float32),
    )(p0, p1, g, dinv, b1, W2, b2, W3, b3, Wl, bl)


def kernel(x, edge_index, W1, b1, W2, b2, W3, b3, Wl, bl):
    ei = edge_index.astype(jnp.int32)
    src3 = ei[0].reshape(NW, NCH, CH)
    dst3 = ei[1].reshape(NW, NCH, CH)
    zeros1 = jnp.zeros((N,), jnp.float32)
    zeros2 = jnp.zeros((N, D), jnp.float32)

    degp = _deg_kernel(dst3, zeros1)
    d0 = degp[0].reshape(N, 1)
    d1 = degp[1].reshape(N, 1)

    g, dinv = _mm1(x, d0, d1, W1)

    parts = _agg_kernel(src3, dst3, g, zeros2)

    return _mlp(parts[0], parts[1], g, dinv,
                b1.reshape(1, D), W2, b2.reshape(1, D),
                W3, b3.reshape(1, D), Wl, bl.reshape(1, OUT))


# speedup vs baseline: 26.6180x; 26.6180x over previous
import functools

import jax
import jax.numpy as jnp
from jax import lax
from jax.experimental import pallas as pl
from jax.experimental.pallas import tpu as pltpu
from jax.experimental.pallas import tpu_sc as plsc

N = 10000
D = 128
E = 320000
OUT = 2

NC = 2
NS = 16
NW = NC * NS
EPW = E // NW
CH = 80
NCH = EPW // CH

BLK = 2000
GRID = N // BLK


def _sc_mesh():
    return plsc.VectorSubcoreMesh(core_axis_name="c", subcore_axis_name="s")


def _deg_body(dst_hbm, zeros_hbm, out_hbm, idx_v, ones_v, acc_s):
    cid = lax.axis_index("c")
    sid = lax.axis_index("s")
    wid = cid * NS + sid
    for j in range(CH // 16):
        ones_v[pl.ds(j * 16, 16)] = jnp.ones((16,), jnp.float32)

    @pl.when(sid == 0)
    def _():
        pltpu.sync_copy(zeros_hbm, acc_s)

    plsc.subcore_barrier()
    pltpu.sync_copy(dst_hbm.at[wid], idx_v)

    def body(j, carry):
        pltpu.sync_copy(ones_v, acc_s.at[idx_v.at[j]], add=True)
        return carry

    lax.fori_loop(0, NCH, body, 0)
    plsc.subcore_barrier()

    @pl.when(sid == 0)
    def _():
        pltpu.sync_copy(acc_s, out_hbm.at[cid])


_deg_kernel = functools.partial(
    pl.kernel,
    mesh=_sc_mesh(),
    out_type=jax.ShapeDtypeStruct((NC, N), jnp.float32),
    scratch_types=[
        pltpu.VMEM((NCH, CH), jnp.int32),
        pltpu.VMEM((CH,), jnp.float32),
        pltpu.VMEM_SHARED((N,), jnp.float32),
    ],
)(_deg_body)


def _agg_body(src_hbm, dst_hbm, g_hbm, zeros_hbm, out_hbm,
              sidx_v, didx_v, rows_v, acc_s, sem):
    cid = lax.axis_index("c")
    sid = lax.axis_index("s")
    wid = cid * NS + sid
    rows_per = 624
    tail = N - NS * rows_per

    pltpu.sync_copy(zeros_hbm.at[pl.ds(sid * rows_per, rows_per)],
                    acc_s.at[pl.ds(sid * rows_per, rows_per)])

    @pl.when(sid == NS - 1)
    def _():
        pltpu.sync_copy(zeros_hbm.at[pl.ds(NS * rows_per, tail)],
                        acc_s.at[pl.ds(NS * rows_per, tail)])

    plsc.subcore_barrier()

    pltpu.sync_copy(src_hbm.at[wid], sidx_v)
    pltpu.sync_copy(dst_hbm.at[wid], didx_v)

    def body(j, carry):
        pltpu.async_copy(g_hbm.at[sidx_v.at[j]], rows_v, sem).wait()
        pltpu.sync_copy(rows_v, acc_s.at[didx_v.at[j]], add=True)
        return carry

    lax.fori_loop(0, NCH, body, 0)
    plsc.subcore_barrier()

    pltpu.sync_copy(acc_s.at[pl.ds(sid * rows_per, rows_per)],
                    out_hbm.at[cid, pl.ds(sid * rows_per, rows_per)])

    @pl.when(sid == NS - 1)
    def _():
        pltpu.sync_copy(acc_s.at[pl.ds(NS * rows_per, tail)],
                        out_hbm.at[cid, pl.ds(NS * rows_per, tail)])


_agg_kernel = functools.partial(
    pl.kernel,
    mesh=_sc_mesh(),
    out_type=jax.ShapeDtypeStruct((NC, N, D), jnp.float32),
    scratch_types=[
        pltpu.VMEM((NCH, CH), jnp.int32),
        pltpu.VMEM((NCH, CH), jnp.int32),
        pltpu.VMEM((CH, D), jnp.float32),
        pltpu.VMEM_SHARED((N, D), jnp.float32),
        pltpu.SemaphoreType.DMA,
    ],
)(_agg_body)


def _mm1_body(x_ref, d0_ref, d1_ref, w1_ref, g_ref, dinv_ref):
    deg = d0_ref[...] + d1_ref[...] + 1.0
    dinv = lax.rsqrt(deg)
    h = jnp.dot(x_ref[...], w1_ref[...], preferred_element_type=jnp.float32)
    g_ref[...] = h * dinv
    dinv_ref[...] = dinv


def _mm1(x, d0, d1, W1):
    return pl.pallas_call(
        _mm1_body,
        grid=(GRID,),
        in_specs=[
            pl.BlockSpec((BLK, D), lambda i: (i, 0)),
            pl.BlockSpec((BLK, 1), lambda i: (i, 0)),
            pl.BlockSpec((BLK, 1), lambda i: (i, 0)),
            pl.BlockSpec((D, D), lambda i: (0, 0)),
        ],
        out_specs=[
            pl.BlockSpec((BLK, D), lambda i: (i, 0)),
            pl.BlockSpec((BLK, 1), lambda i: (i, 0)),
        ],
        out_shape=[
            jax.ShapeDtypeStruct((N, D), jnp.float32),
            jax.ShapeDtypeStruct((N, 1), jnp.float32),
        ],
    )(x, d0, d1, W1)


def _mlp_body(p0_ref, p1_ref, g_ref, dinv_ref, b1_ref, w2_ref, b2_ref,
              w3_ref, b3_ref, wl_ref, bl_ref, out_ref):
    conv = (p0_ref[...] + p1_ref[...] + g_ref[...]) * dinv_ref[...] + b1_ref[...]
    h = jnp.maximum(conv, 0.0)
    h = jnp.dot(h, w2_ref[...], preferred_element_type=jnp.float32) + b2_ref[...]
    h = jnp.maximum(h, 0.0)
    h = jnp.dot(h, w3_ref[...], preferred_element_type=jnp.float32) + b3_ref[...]
    h = jnp.maximum(h, 0.0)
    out_ref[...] = (
        jnp.dot(h, wl_ref[...], preferred_element_type=jnp.float32) + bl_ref[...]
    )


def _mlp(p0, p1, g, dinv, b1, W2, b2, W3, b3, Wl, bl):
    return pl.pallas_call(
        _mlp_body,
        grid=(GRID,),
        in_specs=[
            pl.BlockSpec((BLK, D), lambda i: (i, 0)),
            pl.BlockSpec((BLK, D), lambda i: (i, 0)),
            pl.BlockSpec((BLK, D), lambda i: (i, 0)),
            pl.BlockSpec((BLK, 1), lambda i: (i, 0)),
            pl.BlockSpec((1, D), lambda i: (0, 0)),
            pl.BlockSpec((D, D), lambda i: (0, 0)),
            pl.BlockSpec((1, D), lambda i: (0, 0)),
            pl.BlockSpec((D, D), lambda i: (0, 0)),
            pl.BlockSpec((1, D), lambda i: (0, 0)),
            pl.BlockSpec((D, OUT), lambda i: (0, 0)),
            pl.BlockSpec((1, OUT), lambda i: (0, 0)),
        ],
        out_specs=pl.BlockSpec((BLK, OUT), lambda i: (i, 0)),
        out_shape=jax.ShapeDtypeStruct((N, OUT), jnp.float32),
    )(p0, p1, g, dinv, b1, W2, b2, W3, b3, Wl, bl)


def kernel(x, edge_index, W1, b1, W2, b2, W3, b3, Wl, bl):
    ei = edge_index.astype(jnp.int32)
    src3 = ei[0].reshape(NW, NCH, CH)
    dst3 = ei[1].reshape(NW, NCH, CH)
    zeros1 = jnp.zeros((N,), jnp.float32)
    zeros2 = jnp.zeros((N, D), jnp.float32)

    degp = _deg_kernel(dst3, zeros1)
    d0 = degp[0].reshape(N, 1)
    d1 = degp[1].reshape(N, 1)

    g, dinv = _mm1(x, d0, d1, W1)

    parts = _agg_kernel(src3, dst3, g, zeros2)

    return _mlp(parts[0], parts[1], g, dinv,
                b1.reshape(1, D), W2, b2.reshape(1, D),
                W3, b3.reshape(1, D), Wl, bl.reshape(1, OUT))


# speedup vs baseline: 37.1320x; 1.3950x over previous
import functools

import jax
import jax.numpy as jnp
from jax import lax
from jax.experimental import pallas as pl
from jax.experimental.pallas import tpu as pltpu
from jax.experimental.pallas import tpu_sc as plsc

N = 10000
D = 128
E = 320000
OUT = 2

NC = 2
NS = 16
NW = NC * NS
EPW = E // NW
CH = 80
NCH = EPW // CH
HALF = 64

BLK = 2000
GRID = N // BLK


def _sc_mesh():
    return plsc.VectorSubcoreMesh(core_axis_name="c", subcore_axis_name="s")


DEG_BLKS = 5
DEG_ROWS = NCH // DEG_BLKS


def _deg_body(dst_hbm, zeros_hbm, out_hbm, idx_v, ones_v, acc_s):
    cid = lax.axis_index("c")
    sid = lax.axis_index("s")
    wid = cid * NS + sid
    for j in range(CH // 16):
        ones_v[pl.ds(j * 16, 16)] = jnp.ones((16,), jnp.float32)

    @pl.when(sid == 0)
    def _():
        pltpu.sync_copy(zeros_hbm, acc_s)

    plsc.subcore_barrier()

    def inner(j, carry):
        pltpu.sync_copy(ones_v, acc_s.at[idx_v.at[j]], add=True)
        return carry

    for b in range(DEG_BLKS):
        pltpu.sync_copy(dst_hbm.at[wid, b], idx_v)
        lax.fori_loop(0, DEG_ROWS, inner, 0)

    plsc.subcore_barrier()

    @pl.when(sid == 0)
    def _():
        pltpu.sync_copy(acc_s, out_hbm.at[cid])


_deg_kernel = functools.partial(
    pl.kernel,
    mesh=_sc_mesh(),
    out_type=jax.ShapeDtypeStruct((NC, N), jnp.float32),
    scratch_types=[
        pltpu.VMEM((DEG_ROWS, CH), jnp.int32),
        pltpu.VMEM((CH,), jnp.float32),
        pltpu.VMEM_SHARED((N,), jnp.float32),
    ],
)(_deg_body)


def _agg_body(src_hbm, dst_hbm, g_hbm, zeros_hbm, out_hbm,
              sidx_v, didx_v, rows_a, rows_b, acc_s, sem_a, sem_b):
    cid = lax.axis_index("c")
    sid = lax.axis_index("s")
    wid = cid * NS + sid
    rows_per = 624
    tail = N - NS * rows_per

    init = (g_hbm, zeros_hbm)

    @pl.when(cid == 0)
    def _():
        pltpu.sync_copy(init[0].at[pl.ds(sid * rows_per, rows_per)],
                        acc_s.at[pl.ds(sid * rows_per, rows_per)])

    @pl.when(cid == 1)
    def _():
        pltpu.sync_copy(init[1].at[pl.ds(sid * rows_per, rows_per)],
                        acc_s.at[pl.ds(sid * rows_per, rows_per)])

    @pl.when((sid == NS - 1) & (cid == 0))
    def _():
        pltpu.sync_copy(init[0].at[pl.ds(NS * rows_per, tail)],
                        acc_s.at[pl.ds(NS * rows_per, tail)])

    @pl.when((sid == NS - 1) & (cid == 1))
    def _():
        pltpu.sync_copy(init[1].at[pl.ds(NS * rows_per, tail)],
                        acc_s.at[pl.ds(NS * rows_per, tail)])

    plsc.subcore_barrier()

    def gather(j, buf, sem):
        pltpu.async_copy(g_hbm.at[sidx_v.at[j]], buf, sem)

    def wait_gather(buf, sem):
        pltpu.make_async_copy(g_hbm.at[sidx_v.at[0]], buf, sem).wait()

    def run_block(nrows):
        gather(0, rows_a, sem_a)
        gather(1, rows_b, sem_b)

        def body(k, carry):
            j0 = 2 * k
            j1 = j0 + 1
            wait_gather(rows_a, sem_a)
            pltpu.sync_copy(rows_a, acc_s.at[didx_v.at[j0]], add=True)

            @pl.when(j0 + 2 < nrows)
            def _():
                gather(j0 + 2, rows_a, sem_a)

            wait_gather(rows_b, sem_b)
            pltpu.sync_copy(rows_b, acc_s.at[didx_v.at[j1]], add=True)

            @pl.when(j1 + 2 < nrows)
            def _():
                gather(j1 + 2, rows_b, sem_b)

            return carry

        lax.fori_loop(0, nrows // 2, body, 0)
        if nrows % 2:
            wait_gather(rows_a, sem_a)
            pltpu.sync_copy(rows_a, acc_s.at[didx_v.at[nrows - 1]], add=True)

    for r0, nrows in ((0, HALF), (HALF, NCH - HALF)):
        pltpu.sync_copy(src_hbm.at[wid, pl.ds(r0, nrows)],
                        sidx_v.at[pl.ds(0, nrows)])
        pltpu.sync_copy(dst_hbm.at[wid, pl.ds(r0, nrows)],
                        didx_v.at[pl.ds(0, nrows)])
        run_block(nrows)

    plsc.subcore_barrier()

    pltpu.sync_copy(acc_s.at[pl.ds(sid * rows_per, rows_per)],
                    out_hbm.at[cid, pl.ds(sid * rows_per, rows_per)])

    @pl.when(sid == NS - 1)
    def _():
        pltpu.sync_copy(acc_s.at[pl.ds(NS * rows_per, tail)],
                        out_hbm.at[cid, pl.ds(NS * rows_per, tail)])


_agg_kernel = functools.partial(
    pl.kernel,
    mesh=_sc_mesh(),
    out_type=jax.ShapeDtypeStruct((NC, N, D), jnp.float32),
    scratch_types=[
        pltpu.VMEM((HALF, CH), jnp.int32),
        pltpu.VMEM((HALF, CH), jnp.int32),
        pltpu.VMEM((CH, D), jnp.float32),
        pltpu.VMEM((CH, D), jnp.float32),
        pltpu.VMEM_SHARED((N, D), jnp.float32),
        pltpu.SemaphoreType.DMA,
        pltpu.SemaphoreType.DMA,
    ],
)(_agg_body)


def _mm1_body(x_ref, d0_ref, d1_ref, w1_ref, g_ref, dinv_ref):
    deg = d0_ref[...] + d1_ref[...] + 1.0
    dinv = lax.rsqrt(deg)
    h = jnp.dot(x_ref[...], w1_ref[...], preferred_element_type=jnp.float32)
    g_ref[...] = h * dinv
    dinv_ref[...] = dinv


def _mm1(x, d0, d1, W1):
    return pl.pallas_call(
        _mm1_body,
        grid=(GRID,),
        in_specs=[
            pl.BlockSpec((BLK, D), lambda i: (i, 0)),
            pl.BlockSpec((BLK, 1), lambda i: (i, 0)),
            pl.BlockSpec((BLK, 1), lambda i: (i, 0)),
            pl.BlockSpec((D, D), lambda i: (0, 0)),
        ],
        out_specs=[
            pl.BlockSpec((BLK, D), lambda i: (i, 0)),
            pl.BlockSpec((BLK, 1), lambda i: (i, 0)),
        ],
        out_shape=[
            jax.ShapeDtypeStruct((N, D), jnp.float32),
            jax.ShapeDtypeStruct((N, 1), jnp.float32),
        ],
    )(x, d0, d1, W1)


def _mlp_body(p0_ref, p1_ref, dinv_ref, b1_ref, w2_ref, b2_ref,
              w3_ref, b3_ref, wl_ref, bl_ref, out_ref):
    conv = (p0_ref[...] + p1_ref[...]) * dinv_ref[...] + b1_ref[...]
    h = jnp.maximum(conv, 0.0)
    h = jnp.dot(h, w2_ref[...], preferred_element_type=jnp.float32) + b2_ref[...]
    h = jnp.maximum(h, 0.0)
    h = jnp.dot(h, w3_ref[...], preferred_element_type=jnp.float32) + b3_ref[...]
    h = jnp.maximum(h, 0.0)
    out_ref[...] = (
        jnp.dot(h, wl_ref[...], preferred_element_type=jnp.float32) + bl_ref[...]
    )


def _mlp(p0, p1, dinv, b1, W2, b2, W3, b3, Wl, bl):
    return pl.pallas_call(
        _mlp_body,
        grid=(GRID,),
        in_specs=[
            pl.BlockSpec((BLK, D), lambda i: (i, 0)),
            pl.BlockSpec((BLK, D), lambda i: (i, 0)),
            pl.BlockSpec((BLK, 1), lambda i: (i, 0)),
            pl.BlockSpec((1, D), lambda i: (0, 0)),
            pl.BlockSpec((D, D), lambda i: (0, 0)),
            pl.BlockSpec((1, D), lambda i: (0, 0)),
            pl.BlockSpec((D, D), lambda i: (0, 0)),
            pl.BlockSpec((1, D), lambda i: (0, 0)),
            pl.BlockSpec((D, OUT), lambda i: (0, 0)),
            pl.BlockSpec((1, OUT), lambda i: (0, 0)),
        ],
        out_specs=pl.BlockSpec((BLK, OUT), lambda i: (i, 0)),
        out_shape=jax.ShapeDtypeStruct((N, OUT), jnp.float32),
    )(p0, p1, dinv, b1, W2, b2, W3, b3, Wl, bl)


def kernel(x, edge_index, W1, b1, W2, b2, W3, b3, Wl, bl):
    ei = edge_index.astype(jnp.int32)
    src3 = ei[0].reshape(NW, NCH, CH)
    dst3 = ei[1].reshape(NW, NCH, CH)
    dst4 = ei[1].reshape(NW, DEG_BLKS, DEG_ROWS, CH)
    zeros1 = jnp.zeros((N,), jnp.float32)
    zeros2 = jnp.zeros((N, D), jnp.float32)

    degp = _deg_kernel(dst4, zeros1)
    d0 = degp[0].reshape(N, 1)
    d1 = degp[1].reshape(N, 1)

    g, dinv = _mm1(x, d0, d1, W1)

    parts = _agg_kernel(src3, dst3, g, zeros2)

    return _mlp(parts[0], parts[1], dinv,
                b1.reshape(1, D), W2, b2.reshape(1, D),
                W3, b3.reshape(1, D), Wl, bl.reshape(1, OUT))


# speedup vs baseline: 39.0798x; 1.0525x over previous
import functools

import jax
import jax.numpy as jnp
from jax import lax
from jax.experimental import pallas as pl
from jax.experimental.pallas import tpu as pltpu
from jax.experimental.pallas import tpu_sc as plsc

N = 10000
D = 128
E = 320000
OUT = 2

NC = 2
NS = 16
NW = NC * NS
EPW = E // NW
CH = 128
PAD = 240
NCH = (EPW + PAD) // CH
IH = 40
JUNK = 64
NACC = N + JUNK

SLAB = 624

BLK = 2000
GRID = N // BLK


def _sc_mesh():
    return plsc.VectorSubcoreMesh(core_axis_name="c", subcore_axis_name="s")


def _deg_body(dst_hbm, zeros_hbm, out_hbm, idx_v, ones_v, acc_s):
    cid = lax.axis_index("c")
    sid = lax.axis_index("s")
    wid = cid * NS + sid
    for j in range(CH // 16):
        ones_v[pl.ds(j * 16, 16)] = jnp.ones((16,), jnp.float32)

    @pl.when(sid == 0)
    def _():
        pltpu.sync_copy(zeros_hbm, acc_s)

    plsc.subcore_barrier()

    def inner(j, carry):
        pltpu.sync_copy(ones_v, acc_s.at[idx_v.at[j]], add=True)
        return carry

    for r0 in range(0, NCH, IH):
        pltpu.sync_copy(dst_hbm.at[wid, pl.ds(r0, IH)], idx_v)
        lax.fori_loop(0, IH, inner, 0)

    plsc.subcore_barrier()

    @pl.when(sid == 0)
    def _():
        pltpu.sync_copy(acc_s, out_hbm.at[cid])


_deg_kernel = functools.partial(
    pl.kernel,
    mesh=_sc_mesh(),
    out_type=jax.ShapeDtypeStruct((NC, NACC), jnp.float32),
    scratch_types=[
        pltpu.VMEM((IH, CH), jnp.int32),
        pltpu.VMEM((CH,), jnp.float32),
        pltpu.VMEM_SHARED((NACC,), jnp.float32),
    ],
)(_deg_body)


def _agg_body(src_hbm, dst_hbm, g_hbm, zeros_hbm, out_hbm,
              sidx_v, didx_v, rows_a, rows_b, acc_s, sem_a, sem_b):
    cid = lax.axis_index("c")
    sid = lax.axis_index("s")
    wid = cid * NS + sid
    tail = N - NS * SLAB

    @pl.when(cid == 0)
    def _():
        pltpu.sync_copy(g_hbm.at[pl.ds(sid * SLAB, SLAB)],
                        acc_s.at[pl.ds(sid * SLAB, SLAB)])

        @pl.when(sid == NS - 1)
        def _():
            pltpu.sync_copy(g_hbm.at[pl.ds(NS * SLAB, tail)],
                            acc_s.at[pl.ds(NS * SLAB, tail)])
            pltpu.sync_copy(zeros_hbm.at[pl.ds(N, JUNK)],
                            acc_s.at[pl.ds(N, JUNK)])

    @pl.when(cid == 1)
    def _():
        pltpu.sync_copy(zeros_hbm.at[pl.ds(sid * SLAB, SLAB)],
                        acc_s.at[pl.ds(sid * SLAB, SLAB)])

        @pl.when(sid == NS - 1)
        def _():
            pltpu.sync_copy(zeros_hbm.at[pl.ds(NS * SLAB, tail + JUNK)],
                            acc_s.at[pl.ds(NS * SLAB, tail + JUNK)])

    plsc.subcore_barrier()

    def gather(j, buf, sem):
        pltpu.async_copy(g_hbm.at[sidx_v.at[j]], buf, sem)

    def wait_gather(buf, sem):
        pltpu.make_async_copy(g_hbm.at[sidx_v.at[0]], buf, sem).wait()

    def run_block(nrows):
        gather(0, rows_a, sem_a)
        gather(1, rows_b, sem_b)

        def body(k, carry):
            j0 = 2 * k
            j1 = j0 + 1
            wait_gather(rows_a, sem_a)
            pltpu.sync_copy(rows_a, acc_s.at[didx_v.at[j0]], add=True)

            @pl.when(j0 + 2 < nrows)
            def _():
                gather(j0 + 2, rows_a, sem_a)

            wait_gather(rows_b, sem_b)
            pltpu.sync_copy(rows_b, acc_s.at[didx_v.at[j1]], add=True)

            @pl.when(j1 + 2 < nrows)
            def _():
                gather(j1 + 2, rows_b, sem_b)

            return carry

        lax.fori_loop(0, nrows // 2, body, 0)

    for r0 in range(0, NCH, IH):
        pltpu.sync_copy(src_hbm.at[wid, pl.ds(r0, IH)], sidx_v)
        pltpu.sync_copy(dst_hbm.at[wid, pl.ds(r0, IH)], didx_v)
        run_block(IH)

    plsc.subcore_barrier()

    pltpu.sync_copy(acc_s.at[pl.ds(sid * SLAB, SLAB)],
                    out_hbm.at[cid, pl.ds(sid * SLAB, SLAB)])

    @pl.when(sid == NS - 1)
    def _():
        pltpu.sync_copy(acc_s.at[pl.ds(NS * SLAB, tail)],
                        out_hbm.at[cid, pl.ds(NS * SLAB, tail)])


_agg_kernel = functools.partial(
    pl.kernel,
    mesh=_sc_mesh(),
    out_type=jax.ShapeDtypeStruct((NC, N, D), jnp.float32),
    scratch_types=[
        pltpu.VMEM((IH, CH), jnp.int32),
        pltpu.VMEM((IH, CH), jnp.int32),
        pltpu.VMEM((CH, D), jnp.float32),
        pltpu.VMEM((CH, D), jnp.float32),
        pltpu.VMEM_SHARED((NACC, D), jnp.float32),
        pltpu.SemaphoreType.DMA,
        pltpu.SemaphoreType.DMA,
    ],
)(_agg_body)


def _mm1_body(x_ref, d0_ref, d1_ref, w1_ref, g_ref, dinv_ref):
    deg = d0_ref[...] + d1_ref[...] + 1.0
    dinv = lax.rsqrt(deg)
    h = jnp.dot(x_ref[...], w1_ref[...], preferred_element_type=jnp.float32)
    g_ref[...] = h * dinv
    dinv_ref[...] = dinv


def _mm1(x, d0, d1, W1):
    return pl.pallas_call(
        _mm1_body,
        grid=(GRID,),
        in_specs=[
            pl.BlockSpec((BLK, D), lambda i: (i, 0)),
            pl.BlockSpec((BLK, 1), lambda i: (i, 0)),
            pl.BlockSpec((BLK, 1), lambda i: (i, 0)),
            pl.BlockSpec((D, D), lambda i: (0, 0)),
        ],
        out_specs=[
            pl.BlockSpec((BLK, D), lambda i: (i, 0)),
            pl.BlockSpec((BLK, 1), lambda i: (i, 0)),
        ],
        out_shape=[
            jax.ShapeDtypeStruct((N, D), jnp.float32),
            jax.ShapeDtypeStruct((N, 1), jnp.float32),
        ],
    )(x, d0, d1, W1)


def _mlp_body(p0_ref, p1_ref, dinv_ref, b1_ref, w2_ref, b2_ref,
              w3_ref, b3_ref, wl_ref, bl_ref, out_ref):
    conv = (p0_ref[...] + p1_ref[...]) * dinv_ref[...] + b1_ref[...]
    h = jnp.maximum(conv, 0.0)
    h = jnp.dot(h, w2_ref[...], preferred_element_type=jnp.float32) + b2_ref[...]
    h = jnp.maximum(h, 0.0)
    h = jnp.dot(h, w3_ref[...], preferred_element_type=jnp.float32) + b3_ref[...]
    h = jnp.maximum(h, 0.0)
    out_ref[...] = (
        jnp.dot(h, wl_ref[...], preferred_element_type=jnp.float32) + bl_ref[...]
    )


def _mlp(p0, p1, dinv, b1, W2, b2, W3, b3, Wl, bl):
    return pl.pallas_call(
        _mlp_body,
        grid=(GRID,),
        in_specs=[
            pl.BlockSpec((BLK, D), lambda i: (i, 0)),
            pl.BlockSpec((BLK, D), lambda i: (i, 0)),
            pl.BlockSpec((BLK, 1), lambda i: (i, 0)),
            pl.BlockSpec((1, D), lambda i: (0, 0)),
            pl.BlockSpec((D, D), lambda i: (0, 0)),
            pl.BlockSpec((1, D), lambda i: (0, 0)),
            pl.BlockSpec((D, D), lambda i: (0, 0)),
            pl.BlockSpec((1, D), lambda i: (0, 0)),
            pl.BlockSpec((D, OUT), lambda i: (0, 0)),
            pl.BlockSpec((1, OUT), lambda i: (0, 0)),
        ],
        out_specs=pl.BlockSpec((BLK, OUT), lambda i: (i, 0)),
        out_shape=jax.ShapeDtypeStruct((N, OUT), jnp.float32),
    )(p0, p1, dinv, b1, W2, b2, W3, b3, Wl, bl)


def kernel(x, edge_index, W1, b1, W2, b2, W3, b3, Wl, bl):
    ei = edge_index.astype(jnp.int32)
    padi = jnp.arange(PAD, dtype=jnp.int32) % JUNK
    src_pad = jnp.broadcast_to(padi, (NW, PAD))
    dst_pad = jnp.broadcast_to(N + padi, (NW, PAD))
    src3 = jnp.concatenate(
        [ei[0].reshape(NW, EPW), src_pad], axis=1).reshape(NW, NCH, CH)
    dst3 = jnp.concatenate(
        [ei[1].reshape(NW, EPW), dst_pad], axis=1).reshape(NW, NCH, CH)
    zeros1 = jnp.zeros((NACC,), jnp.float32)
    zeros2 = jnp.zeros((NACC, D), jnp.float32)

    degp = _deg_kernel(dst3, zeros1)
    d0 = degp[0, :N].reshape(N, 1)
    d1 = degp[1, :N].reshape(N, 1)

    g, dinv = _mm1(x, d0, d1, W1)

    parts = _agg_kernel(src3, dst3, g, zeros2)

    return _mlp(parts[0], parts[1], dinv,
                b1.reshape(1, D), W2, b2.reshape(1, D),
                W3, b3.reshape(1, D), Wl, bl.reshape(1, OUT))


# speedup vs baseline: 41.0627x; 1.0507x over previous
import functools

import jax
import jax.numpy as jnp
from jax import lax
from jax.experimental import pallas as pl
from jax.experimental.pallas import tpu as pltpu
from jax.experimental.pallas import tpu_sc as plsc

N = 10000
D = 128
E = 320000
OUT = 2

NC = 2
NS = 16
NW = NC * NS
EPW = E // NW
CH = 128
PAD = 240
NCH = (EPW + PAD) // CH
IH = 40
JUNK = 64
NACC = N + JUNK

SLAB = 624

BLK = 2000
GRID = N // BLK


def _sc_mesh():
    return plsc.VectorSubcoreMesh(core_axis_name="c", subcore_axis_name="s")


def _deg_body(dst_hbm, zeros_hbm, out_hbm, idx_v, ones_v, acc_s):
    cid = lax.axis_index("c")
    sid = lax.axis_index("s")
    wid = cid * NS + sid
    for j in range(CH // 16):
        ones_v[pl.ds(j * 16, 16)] = jnp.ones((16,), jnp.float32)

    @pl.when(sid == 0)
    def _():
        pltpu.sync_copy(zeros_hbm, acc_s)

    plsc.subcore_barrier()

    def inner(j, carry):
        pltpu.sync_copy(ones_v, acc_s.at[idx_v.at[j]], add=True)
        return carry

    for r0 in range(0, NCH, IH):
        pltpu.sync_copy(dst_hbm.at[wid, pl.ds(r0, IH)], idx_v)
        lax.fori_loop(0, IH, inner, 0)

    plsc.subcore_barrier()

    @pl.when(sid == 0)
    def _():
        pltpu.sync_copy(acc_s, out_hbm.at[cid])


_deg_kernel = functools.partial(
    pl.kernel,
    mesh=_sc_mesh(),
    out_type=jax.ShapeDtypeStruct((NC, NACC), jnp.float32),
    scratch_types=[
        pltpu.VMEM((IH, CH), jnp.int32),
        pltpu.VMEM((CH,), jnp.float32),
        pltpu.VMEM_SHARED((NACC,), jnp.float32),
    ],
)(_deg_body)


def _agg_body(src_hbm, dst_hbm, g_hbm, zeros_hbm, out_hbm,
              sidx_v, didx_v, rows_a, rows_b, acc_s, sem_a, sem_b):
    cid = lax.axis_index("c")
    sid = lax.axis_index("s")
    wid = cid * NS + sid
    tail = N - NS * SLAB

    @pl.when(cid == 0)
    def _():
        pltpu.sync_copy(g_hbm.at[pl.ds(sid * SLAB, SLAB)],
                        acc_s.at[pl.ds(sid * SLAB, SLAB)])

        @pl.when(sid == NS - 1)
        def _():
            pltpu.sync_copy(g_hbm.at[pl.ds(NS * SLAB, tail)],
                            acc_s.at[pl.ds(NS * SLAB, tail)])
            pltpu.sync_copy(zeros_hbm.at[pl.ds(N, JUNK)],
                            acc_s.at[pl.ds(N, JUNK)])

    @pl.when(cid == 1)
    def _():
        pltpu.sync_copy(zeros_hbm.at[pl.ds(sid * SLAB, SLAB)],
                        acc_s.at[pl.ds(sid * SLAB, SLAB)])

        @pl.when(sid == NS - 1)
        def _():
            pltpu.sync_copy(zeros_hbm.at[pl.ds(NS * SLAB, tail + JUNK)],
                            acc_s.at[pl.ds(NS * SLAB, tail + JUNK)])

    plsc.subcore_barrier()

    def gather(j, buf, sem):
        pltpu.async_copy(g_hbm.at[sidx_v.at[j]], buf, sem)

    def wait_gather(buf, sem):
        pltpu.make_async_copy(g_hbm.at[sidx_v.at[0]], buf, sem).wait()

    def run_block(nrows):
        gather(0, rows_a, sem_a)
        gather(1, rows_b, sem_b)

        def body(k, carry):
            j0 = 2 * k
            j1 = j0 + 1
            wait_gather(rows_a, sem_a)
            pltpu.sync_copy(rows_a, acc_s.at[didx_v.at[j0]], add=True)

            @pl.when(j0 + 2 < nrows)
            def _():
                gather(j0 + 2, rows_a, sem_a)

            wait_gather(rows_b, sem_b)
            pltpu.sync_copy(rows_b, acc_s.at[didx_v.at[j1]], add=True)

            @pl.when(j1 + 2 < nrows)
            def _():
                gather(j1 + 2, rows_b, sem_b)

            return carry

        lax.fori_loop(0, nrows // 2, body, 0)

    for r0 in range(0, NCH, IH):
        pltpu.sync_copy(src_hbm.at[wid, pl.ds(r0, IH)], sidx_v)
        pltpu.sync_copy(dst_hbm.at[wid, pl.ds(r0, IH)], didx_v)
        run_block(IH)

    plsc.subcore_barrier()

    pltpu.sync_copy(acc_s.at[pl.ds(sid * SLAB, SLAB)],
                    out_hbm.at[cid, pl.ds(sid * SLAB, SLAB)])

    @pl.when(sid == NS - 1)
    def _():
        pltpu.sync_copy(acc_s.at[pl.ds(NS * SLAB, tail)],
                        out_hbm.at[cid, pl.ds(NS * SLAB, tail)])


_agg_kernel = functools.partial(
    pl.kernel,
    mesh=_sc_mesh(),
    out_type=jax.ShapeDtypeStruct((NC, N, D), jnp.float32),
    compiler_params=pltpu.CompilerParams(use_tc_tiling_on_sc=True),
    scratch_types=[
        pltpu.VMEM((IH, CH), jnp.int32),
        pltpu.VMEM((IH, CH), jnp.int32),
        pltpu.VMEM((CH, D), jnp.float32),
        pltpu.VMEM((CH, D), jnp.float32),
        pltpu.VMEM_SHARED((NACC, D), jnp.float32),
        pltpu.SemaphoreType.DMA,
        pltpu.SemaphoreType.DMA,
    ],
)(_agg_body)


def _mm1_body(x_ref, d0_ref, d1_ref, w1_ref, g_ref, dinv_ref):
    deg = d0_ref[...] + d1_ref[...] + 1.0
    dinv = lax.rsqrt(deg)
    h = jnp.dot(x_ref[...], w1_ref[...], preferred_element_type=jnp.float32)
    g_ref[...] = h * dinv
    dinv_ref[...] = dinv


def _mm1(x, d0, d1, W1):
    return pl.pallas_call(
        _mm1_body,
        grid=(GRID,),
        in_specs=[
            pl.BlockSpec((BLK, D), lambda i: (i, 0)),
            pl.BlockSpec((BLK, 1), lambda i: (i, 0)),
            pl.BlockSpec((BLK, 1), lambda i: (i, 0)),
            pl.BlockSpec((D, D), lambda i: (0, 0)),
        ],
        out_specs=[
            pl.BlockSpec((BLK, D), lambda i: (i, 0)),
            pl.BlockSpec((BLK, 1), lambda i: (i, 0)),
        ],
        out_shape=[
            jax.ShapeDtypeStruct((N, D), jnp.float32),
            jax.ShapeDtypeStruct((N, 1), jnp.float32),
        ],
    )(x, d0, d1, W1)


def _mlp_body(p0_ref, p1_ref, dinv_ref, b1_ref, w2_ref, b2_ref,
              w3_ref, b3_ref, wl_ref, bl_ref, out_ref):
    conv = (p0_ref[0] + p1_ref[0]) * dinv_ref[...] + b1_ref[...]
    h = jnp.maximum(conv, 0.0)
    h = jnp.dot(h, w2_ref[...], preferred_element_type=jnp.float32) + b2_ref[...]
    h = jnp.maximum(h, 0.0)
    h = jnp.dot(h, w3_ref[...], preferred_element_type=jnp.float32) + b3_ref[...]
    h = jnp.maximum(h, 0.0)
    out_ref[...] = (
        jnp.dot(h, wl_ref[...], preferred_element_type=jnp.float32) + bl_ref[...]
    )


def _mlp(parts, dinv, b1, W2, b2, W3, b3, Wl, bl):
    return pl.pallas_call(
        _mlp_body,
        grid=(GRID,),
        in_specs=[
            pl.BlockSpec((1, BLK, D), lambda i: (0, i, 0)),
            pl.BlockSpec((1, BLK, D), lambda i: (1, i, 0)),
            pl.BlockSpec((BLK, 1), lambda i: (i, 0)),
            pl.BlockSpec((1, D), lambda i: (0, 0)),
            pl.BlockSpec((D, D), lambda i: (0, 0)),
            pl.BlockSpec((1, D), lambda i: (0, 0)),
            pl.BlockSpec((D, D), lambda i: (0, 0)),
            pl.BlockSpec((1, D), lambda i: (0, 0)),
            pl.BlockSpec((D, OUT), lambda i: (0, 0)),
            pl.BlockSpec((1, OUT), lambda i: (0, 0)),
        ],
        out_specs=pl.BlockSpec((BLK, OUT), lambda i: (i, 0)),
        out_shape=jax.ShapeDtypeStruct((N, OUT), jnp.float32),
    )(parts, parts, dinv, b1, W2, b2, W3, b3, Wl, bl)


def kernel(x, edge_index, W1, b1, W2, b2, W3, b3, Wl, bl):
    ei = edge_index.astype(jnp.int32)
    padi = jnp.arange(PAD, dtype=jnp.int32) % JUNK
    src_pad = jnp.broadcast_to(padi, (NW, PAD))
    dst_pad = jnp.broadcast_to(N + padi, (NW, PAD))
    src3 = jnp.concatenate(
        [ei[0].reshape(NW, EPW), src_pad], axis=1).reshape(NW, NCH, CH)
    dst3 = jnp.concatenate(
        [ei[1].reshape(NW, EPW), dst_pad], axis=1).reshape(NW, NCH, CH)
    zeros1 = jnp.zeros((NACC,), jnp.float32)
    zeros2 = jnp.zeros((NACC, D), jnp.float32)

    degp = _deg_kernel(dst3, zeros1)
    d0 = degp[0, :N].reshape(N, 1)
    d1 = degp[1, :N].reshape(N, 1)

    g, dinv = _mm1(x, d0, d1, W1)

    parts = _agg_kernel(src3, dst3, g, zeros2)

    return _mlp(parts, dinv,
                b1.reshape(1, D), W2, b2.reshape(1, D),
                W3, b3.reshape(1, D), Wl, bl.reshape(1, OUT))


# speedup vs baseline: 42.2468x; 1.0288x over previous
import functools

import jax
import jax.numpy as jnp
from jax import lax
from jax.experimental import pallas as pl
from jax.experimental.pallas import tpu as pltpu
from jax.experimental.pallas import tpu_sc as plsc

N = 10000
D = 128
E = 320000
OUT = 2

NC = 2
NS = 16
NW = NC * NS
EPW = E // NW
CH = 128
PAD = 240
NCH = (EPW + PAD) // CH
IH = 40
JUNK = 64
NACC = N + JUNK

SLAB = 624

BLK = 2000
GRID = N // BLK


def _sc_mesh():
    return plsc.VectorSubcoreMesh(core_axis_name="c", subcore_axis_name="s")


def _deg_body(dst_hbm, zeros_hbm, out_hbm, idx_v, ones_v, acc_s):
    cid = lax.axis_index("c")
    sid = lax.axis_index("s")
    wid = cid * NS + sid
    for j in range(CH // 16):
        ones_v[pl.ds(j * 16, 16)] = jnp.ones((16,), jnp.float32)

    @pl.when(sid == 0)
    def _():
        pltpu.sync_copy(zeros_hbm, acc_s)

    plsc.subcore_barrier()

    def inner(j, carry):
        pltpu.sync_copy(ones_v, acc_s.at[idx_v.at[j]], add=True)
        return carry

    for r0 in range(0, NCH, IH):
        pltpu.sync_copy(dst_hbm.at[wid, pl.ds(r0, IH)], idx_v)
        lax.fori_loop(0, IH, inner, 0)

    plsc.subcore_barrier()

    @pl.when(sid == 0)
    def _():
        pltpu.sync_copy(acc_s, out_hbm.at[cid])


_deg_kernel = functools.partial(
    pl.kernel,
    mesh=_sc_mesh(),
    out_type=jax.ShapeDtypeStruct((NC, NACC), jnp.float32),
    scratch_types=[
        pltpu.VMEM((IH, CH), jnp.int32),
        pltpu.VMEM((CH,), jnp.float32),
        pltpu.VMEM_SHARED((NACC,), jnp.float32),
    ],
)(_deg_body)


def _agg_body(src_hbm, dst_hbm, g_hbm, zeros_hbm, out_hbm,
              sidx_v, didx_v, rows_a, rows_b, acc_s, sem_a, sem_b):
    cid = lax.axis_index("c")
    sid = lax.axis_index("s")
    wid = cid * NS + sid
    tail = N - NS * SLAB

    @pl.when(cid == 0)
    def _():
        pltpu.sync_copy(g_hbm.at[pl.ds(sid * SLAB, SLAB)],
                        acc_s.at[pl.ds(sid * SLAB, SLAB)])

        @pl.when(sid == NS - 1)
        def _():
            pltpu.sync_copy(g_hbm.at[pl.ds(NS * SLAB, tail)],
                            acc_s.at[pl.ds(NS * SLAB, tail)])
            pltpu.sync_copy(zeros_hbm.at[pl.ds(N, JUNK)],
                            acc_s.at[pl.ds(N, JUNK)])

    @pl.when(cid == 1)
    def _():
        pltpu.sync_copy(zeros_hbm.at[pl.ds(sid * SLAB, SLAB)],
                        acc_s.at[pl.ds(sid * SLAB, SLAB)])

        @pl.when(sid == NS - 1)
        def _():
            pltpu.sync_copy(zeros_hbm.at[pl.ds(NS * SLAB, tail + JUNK)],
                            acc_s.at[pl.ds(NS * SLAB, tail + JUNK)])

    plsc.subcore_barrier()

    def gather(j, buf, sem):
        pltpu.async_copy(g_hbm.at[sidx_v.at[j]], buf, sem)

    def wait_gather(buf, sem):
        pltpu.make_async_copy(g_hbm.at[sidx_v.at[0]], buf, sem).wait()

    def run_block(nrows):
        gather(0, rows_a, sem_a)
        gather(1, rows_b, sem_b)

        def body(k, carry):
            j0 = 2 * k
            j1 = j0 + 1
            wait_gather(rows_a, sem_a)
            pltpu.sync_copy(rows_a, acc_s.at[didx_v.at[j0]], add=True)

            @pl.when(j0 + 2 < nrows)
            def _():
                gather(j0 + 2, rows_a, sem_a)

            wait_gather(rows_b, sem_b)
            pltpu.sync_copy(rows_b, acc_s.at[didx_v.at[j1]], add=True)

            @pl.when(j1 + 2 < nrows)
            def _():
                gather(j1 + 2, rows_b, sem_b)

            return carry

        lax.fori_loop(0, nrows // 2, body, 0)

    for r0 in range(0, NCH, IH):
        pltpu.sync_copy(src_hbm.at[wid, pl.ds(r0, IH)], sidx_v)
        pltpu.sync_copy(dst_hbm.at[wid, pl.ds(r0, IH)], didx_v)
        run_block(IH)

    plsc.subcore_barrier()

    pltpu.sync_copy(acc_s.at[pl.ds(sid * SLAB, SLAB)],
                    out_hbm.at[cid, pl.ds(sid * SLAB, SLAB)])

    @pl.when(sid == NS - 1)
    def _():
        pltpu.sync_copy(acc_s.at[pl.ds(NS * SLAB, tail)],
                        out_hbm.at[cid, pl.ds(NS * SLAB, tail)])


_agg_kernel = functools.partial(
    pl.kernel,
    mesh=_sc_mesh(),
    out_type=jax.ShapeDtypeStruct((NC, N, D), jnp.float32),
    compiler_params=pltpu.CompilerParams(use_tc_tiling_on_sc=True),
    scratch_types=[
        pltpu.VMEM((IH, CH), jnp.int32),
        pltpu.VMEM((IH, CH), jnp.int32),
        pltpu.VMEM((CH, D), jnp.float32),
        pltpu.VMEM((CH, D), jnp.float32),
        pltpu.VMEM_SHARED((NACC, D), jnp.float32),
        pltpu.SemaphoreType.DMA,
        pltpu.SemaphoreType.DMA,
    ],
)(_agg_body)


def _mm1_body(x_ref, d0_ref, d1_ref, w1_ref, g_ref, dinv_ref):
    deg = d0_ref[0] + d1_ref[0] + 1.0
    dinv_row = lax.rsqrt(deg)
    dinv_col = jnp.reshape(dinv_row, (BLK, 1))
    h = jnp.dot(x_ref[...], w1_ref[...], preferred_element_type=jnp.float32)
    g_ref[...] = h * dinv_col
    dinv_ref[0] = dinv_row


def _mm1(x, d0, d1, W1):
    return pl.pallas_call(
        _mm1_body,
        grid=(GRID,),
        in_specs=[
            pl.BlockSpec((BLK, D), lambda i: (i, 0)),
            pl.BlockSpec((1, 1, BLK), lambda i: (i, 0, 0)),
            pl.BlockSpec((1, 1, BLK), lambda i: (i, 0, 0)),
            pl.BlockSpec((D, D), lambda i: (0, 0)),
        ],
        out_specs=[
            pl.BlockSpec((BLK, D), lambda i: (i, 0)),
            pl.BlockSpec((1, 1, BLK), lambda i: (i, 0, 0)),
        ],
        out_shape=[
            jax.ShapeDtypeStruct((N, D), jnp.float32),
            jax.ShapeDtypeStruct((GRID, 1, BLK), jnp.float32),
        ],
    )(x, d0, d1, W1)


def _mlp_body(p0_ref, p1_ref, dinv_ref, b1_ref, w2_ref, b2_ref,
              w3_ref, b3_ref, wl_ref, bl_ref, out_ref):
    dinv_col = jnp.reshape(dinv_ref[0], (BLK, 1))
    conv = (p0_ref[0] + p1_ref[0]) * dinv_col + b1_ref[...]
    h = jnp.maximum(conv, 0.0)
    h = jnp.dot(h, w2_ref[...], preferred_element_type=jnp.float32) + b2_ref[...]
    h = jnp.maximum(h, 0.0)
    h = jnp.dot(h, w3_ref[...], preferred_element_type=jnp.float32) + b3_ref[...]
    h = jnp.maximum(h, 0.0)
    out_ref[...] = (
        jnp.dot(h, wl_ref[...], preferred_element_type=jnp.float32) + bl_ref[...]
    )


def _mlp(parts, dinv, b1, W2, b2, W3, b3, Wl, bl):
    return pl.pallas_call(
        _mlp_body,
        grid=(GRID,),
        in_specs=[
            pl.BlockSpec((1, BLK, D), lambda i: (0, i, 0)),
            pl.BlockSpec((1, BLK, D), lambda i: (1, i, 0)),
            pl.BlockSpec((1, 1, BLK), lambda i: (i, 0, 0)),
            pl.BlockSpec((1, D), lambda i: (0, 0)),
            pl.BlockSpec((D, D), lambda i: (0, 0)),
            pl.BlockSpec((1, D), lambda i: (0, 0)),
            pl.BlockSpec((D, D), lambda i: (0, 0)),
            pl.BlockSpec((1, D), lambda i: (0, 0)),
            pl.BlockSpec((D, OUT), lambda i: (0, 0)),
            pl.BlockSpec((1, OUT), lambda i: (0, 0)),
        ],
        out_specs=pl.BlockSpec((BLK, OUT), lambda i: (i, 0)),
        out_shape=jax.ShapeDtypeStruct((N, OUT), jnp.float32),
    )(parts, parts, dinv, b1, W2, b2, W3, b3, Wl, bl)


def kernel(x, edge_index, W1, b1, W2, b2, W3, b3, Wl, bl):
    ei = edge_index.astype(jnp.int32)
    padi = jnp.arange(PAD, dtype=jnp.int32) % JUNK
    src_pad = jnp.broadcast_to(padi, (NW, PAD))
    dst_pad = jnp.broadcast_to(N + padi, (NW, PAD))
    src3 = jnp.concatenate(
        [ei[0].reshape(NW, EPW), src_pad], axis=1).reshape(NW, NCH, CH)
    dst3 = jnp.concatenate(
        [ei[1].reshape(NW, EPW), dst_pad], axis=1).reshape(NW, NCH, CH)
    zeros1 = jnp.zeros((NACC,), jnp.float32)
    zeros2 = jnp.zeros((NACC, D), jnp.float32)

    degp = _deg_kernel(dst3, zeros1)
    d0 = degp[0, :N].reshape(GRID, 1, BLK)
    d1 = degp[1, :N].reshape(GRID, 1, BLK)

    g, dinv = _mm1(x, d0, d1, W1)

    parts = _agg_kernel(src3, dst3, g, zeros2)

    return _mlp(parts, dinv,
                b1.reshape(1, D), W2, b2.reshape(1, D),
                W3, b3.reshape(1, D), Wl, bl.reshape(1, OUT))


# speedup vs baseline: 43.5543x; 1.0309x over previous
import functools

import jax
import jax.numpy as jnp
from jax import lax
from jax.experimental import pallas as pl
from jax.experimental.pallas import tpu as pltpu
from jax.experimental.pallas import tpu_sc as plsc

N = 10000
D = 128
E = 320000
OUT = 2

NC = 2
NS = 16
NW = NC * NS
EPW = E // NW
CH = 128
PAD = 240
NCH = (EPW + PAD) // CH
IH = 40
JUNK = 64
NACC = N + JUNK

SLAB = 624

BLK = 2000
GRID = N // BLK


def _sc_mesh():
    return plsc.VectorSubcoreMesh(core_axis_name="c", subcore_axis_name="s")


def _deg_body(dst_hbm, zeros_hbm, out_hbm, idx_v, ones_v, acc_s):
    cid = lax.axis_index("c")
    sid = lax.axis_index("s")
    wid = cid * NS + sid
    for j in range(CH // 16):
        ones_v[pl.ds(j * 16, 16)] = jnp.ones((16,), jnp.float32)

    @pl.when(sid == 0)
    def _():
        pltpu.sync_copy(zeros_hbm, acc_s)

    plsc.subcore_barrier()

    def inner(j, carry):
        pltpu.sync_copy(ones_v, acc_s.at[idx_v.at[j]], add=True)
        return carry

    for r0 in range(0, NCH, IH):
        pltpu.sync_copy(dst_hbm.at[wid, pl.ds(r0, IH)], idx_v)
        lax.fori_loop(0, IH, inner, 0)

    plsc.subcore_barrier()

    @pl.when(sid == 0)
    def _():
        pltpu.sync_copy(acc_s, out_hbm.at[cid])


_deg_kernel = functools.partial(
    pl.kernel,
    mesh=_sc_mesh(),
    out_type=jax.ShapeDtypeStruct((NC, NACC), jnp.float32),
    scratch_types=[
        pltpu.VMEM((IH, CH), jnp.int32),
        pltpu.VMEM((CH,), jnp.float32),
        pltpu.VMEM_SHARED((NACC,), jnp.float32),
    ],
)(_deg_body)


def _agg_body(src_hbm, dst_hbm, g_hbm, zeros_hbm, out_hbm,
              sidx_v, didx_v, rows_a, rows_b, acc_s, sem_a, sem_b):
    cid = lax.axis_index("c")
    sid = lax.axis_index("s")
    wid = cid * NS + sid
    tail = N - NS * SLAB

    @pl.when(cid == 0)
    def _():
        pltpu.sync_copy(g_hbm.at[pl.ds(sid * SLAB, SLAB)],
                        acc_s.at[pl.ds(sid * SLAB, SLAB)])

        @pl.when(sid == NS - 1)
        def _():
            pltpu.sync_copy(g_hbm.at[pl.ds(NS * SLAB, tail)],
                            acc_s.at[pl.ds(NS * SLAB, tail)])
            pltpu.sync_copy(zeros_hbm.at[pl.ds(0, JUNK)],
                            acc_s.at[pl.ds(N, JUNK)])

    @pl.when(cid == 1)
    def _():
        pltpu.sync_copy(zeros_hbm,
                        acc_s.at[pl.ds(sid * SLAB, SLAB)])

        @pl.when(sid == NS - 1)
        def _():
            pltpu.sync_copy(zeros_hbm.at[pl.ds(0, tail + JUNK)],
                            acc_s.at[pl.ds(NS * SLAB, tail + JUNK)])

    plsc.subcore_barrier()

    def gather(j, buf, sem):
        pltpu.async_copy(g_hbm.at[sidx_v.at[j]], buf, sem)

    def wait_gather(buf, sem):
        pltpu.make_async_copy(g_hbm.at[sidx_v.at[0]], buf, sem).wait()

    def run_block(nrows):
        gather(0, rows_a, sem_a)
        gather(1, rows_b, sem_b)

        def body(k, carry):
            j0 = 2 * k
            j1 = j0 + 1
            wait_gather(rows_a, sem_a)
            pltpu.sync_copy(rows_a, acc_s.at[didx_v.at[j0]], add=True)

            @pl.when(j0 + 2 < nrows)
            def _():
                gather(j0 + 2, rows_a, sem_a)

            wait_gather(rows_b, sem_b)
            pltpu.sync_copy(rows_b, acc_s.at[didx_v.at[j1]], add=True)

            @pl.when(j1 + 2 < nrows)
            def _():
                gather(j1 + 2, rows_b, sem_b)

            return carry

        lax.fori_loop(0, nrows // 2, body, 0)

    for r0 in range(0, NCH, IH):
        pltpu.sync_copy(src_hbm.at[wid, pl.ds(r0, IH)], sidx_v)
        pltpu.sync_copy(dst_hbm.at[wid, pl.ds(r0, IH)], didx_v)
        run_block(IH)

    plsc.subcore_barrier()

    pltpu.sync_copy(acc_s.at[pl.ds(sid * SLAB, SLAB)],
                    out_hbm.at[cid, pl.ds(sid * SLAB, SLAB)])

    @pl.when(sid == NS - 1)
    def _():
        pltpu.sync_copy(acc_s.at[pl.ds(NS * SLAB, tail)],
                        out_hbm.at[cid, pl.ds(NS * SLAB, tail)])


_agg_kernel = functools.partial(
    pl.kernel,
    mesh=_sc_mesh(),
    out_type=jax.ShapeDtypeStruct((NC, N, D), jnp.float32),
    compiler_params=pltpu.CompilerParams(use_tc_tiling_on_sc=True),
    scratch_types=[
        pltpu.VMEM((IH, CH), jnp.int32),
        pltpu.VMEM((IH, CH), jnp.int32),
        pltpu.VMEM((CH, D), jnp.float32),
        pltpu.VMEM((CH, D), jnp.float32),
        pltpu.VMEM_SHARED((NACC, D), jnp.float32),
        pltpu.SemaphoreType.DMA,
        pltpu.SemaphoreType.DMA,
    ],
)(_agg_body)


def _mm1_body(x_ref, d0_ref, d1_ref, w1_ref, g_ref, dinv_ref):
    deg = d0_ref[0] + d1_ref[0] + 1.0
    dinv_row = lax.rsqrt(deg)
    dinv_col = jnp.reshape(dinv_row, (BLK, 1))
    h = jnp.dot(x_ref[...], w1_ref[...], preferred_element_type=jnp.float32)
    g_ref[...] = h * dinv_col
    dinv_ref[0] = dinv_row


def _mm1(x, d0, d1, W1):
    return pl.pallas_call(
        _mm1_body,
        grid=(GRID,),
        in_specs=[
            pl.BlockSpec((BLK, D), lambda i: (i, 0)),
            pl.BlockSpec((1, 1, BLK), lambda i: (i, 0, 0)),
            pl.BlockSpec((1, 1, BLK), lambda i: (i, 0, 0)),
            pl.BlockSpec((D, D), lambda i: (0, 0)),
        ],
        out_specs=[
            pl.BlockSpec((BLK, D), lambda i: (i, 0)),
            pl.BlockSpec((1, 1, BLK), lambda i: (i, 0, 0)),
        ],
        out_shape=[
            jax.ShapeDtypeStruct((N, D), jnp.float32),
            jax.ShapeDtypeStruct((GRID, 1, BLK), jnp.float32),
        ],
    )(x, d0, d1, W1)


def _mlp_body(p0_ref, p1_ref, dinv_ref, b1_ref, w2_ref, b2_ref,
              w3_ref, b3_ref, wl_ref, bl_ref, out_ref):
    dinv_col = jnp.reshape(dinv_ref[0], (BLK, 1))
    conv = (p0_ref[0] + p1_ref[0]) * dinv_col + b1_ref[...]
    h = jnp.maximum(conv, 0.0)
    h = jnp.dot(h, w2_ref[...], preferred_element_type=jnp.float32) + b2_ref[...]
    h = jnp.maximum(h, 0.0)
    h = jnp.dot(h, w3_ref[...], preferred_element_type=jnp.float32) + b3_ref[...]
    h = jnp.maximum(h, 0.0)
    out_ref[0] = (
        lax.dot_general(wl_ref[...], h, (((0,), (1,)), ((), ())),
                        preferred_element_type=jnp.float32)
        + bl_ref[...]
    )


def _mlp(parts, dinv, b1, W2, b2, W3, b3, Wl, bl):
    return pl.pallas_call(
        _mlp_body,
        grid=(GRID,),
        in_specs=[
            pl.BlockSpec((1, BLK, D), lambda i: (0, i, 0)),
            pl.BlockSpec((1, BLK, D), lambda i: (1, i, 0)),
            pl.BlockSpec((1, 1, BLK), lambda i: (i, 0, 0)),
            pl.BlockSpec((1, D), lambda i: (0, 0)),
            pl.BlockSpec((D, D), lambda i: (0, 0)),
            pl.BlockSpec((1, D), lambda i: (0, 0)),
            pl.BlockSpec((D, D), lambda i: (0, 0)),
            pl.BlockSpec((1, D), lambda i: (0, 0)),
            pl.BlockSpec((D, OUT), lambda i: (0, 0)),
            pl.BlockSpec((OUT, 1), lambda i: (0, 0)),
        ],
        out_specs=pl.BlockSpec((1, OUT, BLK), lambda i: (i, 0, 0)),
        out_shape=jax.ShapeDtypeStruct((GRID, OUT, BLK), jnp.float32),
    )(parts, parts, dinv, b1, W2, b2, W3, b3, Wl, bl)


def kernel(x, edge_index, W1, b1, W2, b2, W3, b3, Wl, bl):
    ei = edge_index.astype(jnp.int32)
    padi = jnp.arange(PAD, dtype=jnp.int32) % JUNK
    src_pad = jnp.broadcast_to(padi, (NW, PAD))
    dst_pad = jnp.broadcast_to(N + padi, (NW, PAD))
    src3 = jnp.concatenate(
        [ei[0].reshape(NW, EPW), src_pad], axis=1).reshape(NW, NCH, CH)
    dst3 = jnp.concatenate(
        [ei[1].reshape(NW, EPW), dst_pad], axis=1).reshape(NW, NCH, CH)
    zeros1 = jnp.zeros((NACC,), jnp.float32)
    zeros2 = jnp.zeros((SLAB, D), jnp.float32)

    degp = _deg_kernel(dst3, zeros1)
    d0 = degp[0, :N].reshape(GRID, 1, BLK)
    d1 = degp[1, :N].reshape(GRID, 1, BLK)

    g, dinv = _mm1(x, d0, d1, W1)

    parts = _agg_kernel(src3, dst3, g, zeros2)

    logits_t = _mlp(parts, dinv,
                    b1.reshape(1, D), W2, b2.reshape(1, D),
                    W3, b3.reshape(1, D), Wl, bl.reshape(OUT, 1))
    return logits_t.transpose(0, 2, 1).reshape(N, OUT)


# speedup vs baseline: 44.5556x; 1.0230x over previous
import functools

import jax
import jax.numpy as jnp
from jax import lax
from jax.experimental import pallas as pl
from jax.experimental.pallas import tpu as pltpu
from jax.experimental.pallas import tpu_sc as plsc

N = 10000
D = 128
E = 320000
OUT = 2

NC = 2
NS = 16
NW = NC * NS
EPW = E // NW
CH = 128
PAD = 240
NCH = (EPW + PAD) // CH
IH = 40
JUNK = 64
NACC = N + JUNK

SLAB = 624

BLK = 2000
GRID = N // BLK


def _sc_mesh():
    return plsc.VectorSubcoreMesh(core_axis_name="c", subcore_axis_name="s")


def _deg_body(dst_hbm, zeros_hbm, out_hbm, idx_v, ones_v, acc_s):
    cid = lax.axis_index("c")
    sid = lax.axis_index("s")
    wid = cid * NS + sid
    for j in range(CH // 16):
        ones_v[pl.ds(j * 16, 16)] = jnp.ones((16,), jnp.float32)

    @pl.when(sid == 0)
    def _():
        pltpu.sync_copy(zeros_hbm, acc_s)

    plsc.subcore_barrier()

    def inner(j, carry):
        pltpu.sync_copy(ones_v, acc_s.at[idx_v.at[j]], add=True)
        return carry

    for r0 in range(0, NCH, IH):
        pltpu.sync_copy(dst_hbm.at[wid, pl.ds(r0, IH)], idx_v)
        lax.fori_loop(0, IH, inner, 0)

    plsc.subcore_barrier()

    @pl.when(sid == 0)
    def _():
        pltpu.sync_copy(acc_s, out_hbm.at[cid])


_deg_kernel = functools.partial(
    pl.kernel,
    mesh=_sc_mesh(),
    out_type=jax.ShapeDtypeStruct((NC, NACC), jnp.float32),
    scratch_types=[
        pltpu.VMEM((IH, CH), jnp.int32),
        pltpu.VMEM((CH,), jnp.float32),
        pltpu.VMEM_SHARED((NACC,), jnp.float32),
    ],
)(_deg_body)


def _agg_body(src_hbm, dst_hbm, g_hbm, zeros_hbm, out_hbm,
              sidx_v, didx_v, rows_a, rows_b, acc_s, sem_a, sem_b):
    cid = lax.axis_index("c")
    sid = lax.axis_index("s")
    wid = cid * NS + sid
    tail = N - NS * SLAB

    @pl.when(cid == 0)
    def _():
        pltpu.sync_copy(g_hbm.at[pl.ds(sid * SLAB, SLAB)],
                        acc_s.at[pl.ds(sid * SLAB, SLAB)])

        @pl.when(sid == NS - 1)
        def _():
            pltpu.sync_copy(g_hbm.at[pl.ds(NS * SLAB, tail)],
                            acc_s.at[pl.ds(NS * SLAB, tail)])
            pltpu.sync_copy(zeros_hbm.at[pl.ds(0, JUNK)],
                            acc_s.at[pl.ds(N, JUNK)])

    @pl.when(cid == 1)
    def _():
        pltpu.sync_copy(zeros_hbm,
                        acc_s.at[pl.ds(sid * SLAB, SLAB)])

        @pl.when(sid == NS - 1)
        def _():
            pltpu.sync_copy(zeros_hbm.at[pl.ds(0, tail + JUNK)],
                            acc_s.at[pl.ds(NS * SLAB, tail + JUNK)])

    plsc.subcore_barrier()

    def gather(j, buf, sem):
        pltpu.async_copy(g_hbm.at[sidx_v.at[j]], buf, sem)

    def wait_gather(buf, sem):
        pltpu.make_async_copy(g_hbm.at[sidx_v.at[0]], buf, sem).wait()

    def run_block(nrows):
        gather(0, rows_a, sem_a)
        gather(1, rows_b, sem_b)

        def body(k, carry):
            j0 = 2 * k
            j1 = j0 + 1
            wait_gather(rows_a, sem_a)
            pltpu.sync_copy(rows_a, acc_s.at[didx_v.at[j0]], add=True)

            @pl.when(j0 + 2 < nrows)
            def _():
                gather(j0 + 2, rows_a, sem_a)

            wait_gather(rows_b, sem_b)
            pltpu.sync_copy(rows_b, acc_s.at[didx_v.at[j1]], add=True)

            @pl.when(j1 + 2 < nrows)
            def _():
                gather(j1 + 2, rows_b, sem_b)

            return carry

        lax.fori_loop(0, nrows // 2, body, 0)

    for r0 in range(0, NCH, IH):
        pltpu.sync_copy(src_hbm.at[wid, pl.ds(r0, IH)], sidx_v)
        pltpu.sync_copy(dst_hbm.at[wid, pl.ds(r0, IH)], didx_v)
        run_block(IH)

    plsc.subcore_barrier()

    pltpu.sync_copy(acc_s.at[pl.ds(sid * SLAB, SLAB)],
                    out_hbm.at[cid, pl.ds(sid * SLAB, SLAB)])

    @pl.when(sid == NS - 1)
    def _():
        pltpu.sync_copy(acc_s.at[pl.ds(NS * SLAB, tail)],
                        out_hbm.at[cid, pl.ds(NS * SLAB, tail)])


_agg_kernel = functools.partial(
    pl.kernel,
    mesh=_sc_mesh(),
    out_type=jax.ShapeDtypeStruct((NC, N, D), jnp.float32),
    compiler_params=pltpu.CompilerParams(use_tc_tiling_on_sc=True),
    scratch_types=[
        pltpu.VMEM((IH, CH), jnp.int32),
        pltpu.VMEM((IH, CH), jnp.int32),
        pltpu.VMEM((CH, D), jnp.float32),
        pltpu.VMEM((CH, D), jnp.float32),
        pltpu.VMEM_SHARED((NACC, D), jnp.float32),
        pltpu.SemaphoreType.DMA,
        pltpu.SemaphoreType.DMA,
    ],
)(_agg_body)


def _mmh_body(x_ref, w1_ref, h_ref):
    h_ref[...] = jnp.dot(x_ref[...], w1_ref[...],
                         preferred_element_type=jnp.float32)


def _mmh(x, W1):
    return pl.pallas_call(
        _mmh_body,
        grid=(GRID,),
        in_specs=[
            pl.BlockSpec((BLK, D), lambda i: (i, 0)),
            pl.BlockSpec((D, D), lambda i: (0, 0)),
        ],
        out_specs=pl.BlockSpec((BLK, D), lambda i: (i, 0)),
        out_shape=jax.ShapeDtypeStruct((N, D), jnp.float32),
    )(x, W1)


def _scale_body(h_ref, d0_ref, d1_ref, g_ref, dinv_ref):
    deg = d0_ref[0] + d1_ref[0] + 1.0
    dinv_row = lax.rsqrt(deg)
    dinv_col = jnp.reshape(dinv_row, (BLK, 1))
    g_ref[...] = h_ref[...] * dinv_col
    dinv_ref[0] = dinv_row


def _scale(h, d0, d1):
    return pl.pallas_call(
        _scale_body,
        grid=(GRID,),
        in_specs=[
            pl.BlockSpec((BLK, D), lambda i: (i, 0)),
            pl.BlockSpec((1, 1, BLK), lambda i: (i, 0, 0)),
            pl.BlockSpec((1, 1, BLK), lambda i: (i, 0, 0)),
        ],
        out_specs=[
            pl.BlockSpec((BLK, D), lambda i: (i, 0)),
            pl.BlockSpec((1, 1, BLK), lambda i: (i, 0, 0)),
        ],
        out_shape=[
            jax.ShapeDtypeStruct((N, D), jnp.float32),
            jax.ShapeDtypeStruct((GRID, 1, BLK), jnp.float32),
        ],
    )(h, d0, d1)


def _mlp_body(p0_ref, p1_ref, dinv_ref, b1_ref, w2_ref, b2_ref,
              w3_ref, b3_ref, wl_ref, bl_ref, out_ref):
    dinv_col = jnp.reshape(dinv_ref[0], (BLK, 1))
    conv = (p0_ref[0] + p1_ref[0]) * dinv_col + b1_ref[...]
    h = jnp.maximum(conv, 0.0)
    h = jnp.dot(h, w2_ref[...], preferred_element_type=jnp.float32) + b2_ref[...]
    h = jnp.maximum(h, 0.0)
    h = jnp.dot(h, w3_ref[...], preferred_element_type=jnp.float32) + b3_ref[...]
    h = jnp.maximum(h, 0.0)
    out_ref[0] = (
        lax.dot_general(wl_ref[...], h, (((0,), (1,)), ((), ())),
                        preferred_element_type=jnp.float32)
        + bl_ref[...]
    )


def _mlp(parts, dinv, b1, W2, b2, W3, b3, Wl, bl):
    return pl.pallas_call(
        _mlp_body,
        grid=(GRID,),
        in_specs=[
            pl.BlockSpec((1, BLK, D), lambda i: (0, i, 0)),
            pl.BlockSpec((1, BLK, D), lambda i: (1, i, 0)),
            pl.BlockSpec((1, 1, BLK), lambda i: (i, 0, 0)),
            pl.BlockSpec((1, D), lambda i: (0, 0)),
            pl.BlockSpec((D, D), lambda i: (0, 0)),
            pl.BlockSpec((1, D), lambda i: (0, 0)),
            pl.BlockSpec((D, D), lambda i: (0, 0)),
            pl.BlockSpec((1, D), lambda i: (0, 0)),
            pl.BlockSpec((D, OUT), lambda i: (0, 0)),
            pl.BlockSpec((OUT, 1), lambda i: (0, 0)),
        ],
        out_specs=pl.BlockSpec((1, OUT, BLK), lambda i: (i, 0, 0)),
        out_shape=jax.ShapeDtypeStruct((GRID, OUT, BLK), jnp.float32),
    )(parts, parts, dinv, b1, W2, b2, W3, b3, Wl, bl)


def kernel(x, edge_index, W1, b1, W2, b2, W3, b3, Wl, bl):
    ei = edge_index.astype(jnp.int32)
    padi = jnp.arange(PAD, dtype=jnp.int32) % JUNK
    src_pad = jnp.broadcast_to(padi, (NW, PAD))
    dst_pad = jnp.broadcast_to(N + padi, (NW, PAD))
    src3 = jnp.concatenate(
        [ei[0].reshape(NW, EPW), src_pad], axis=1).reshape(NW, NCH, CH)
    dst3 = jnp.concatenate(
        [ei[1].reshape(NW, EPW), dst_pad], axis=1).reshape(NW, NCH, CH)
    zeros1 = jnp.zeros((NACC,), jnp.float32)
    zeros2 = jnp.zeros((SLAB, D), jnp.float32)

    h = _mmh(x, W1)
    degp = _deg_kernel(dst3, zeros1)
    d0 = degp[0, :N].reshape(GRID, 1, BLK)
    d1 = degp[1, :N].reshape(GRID, 1, BLK)

    g, dinv = _scale(h, d0, d1)

    parts = _agg_kernel(src3, dst3, g, zeros2)

    logits_t = _mlp(parts, dinv,
                    b1.reshape(1, D), W2, b2.reshape(1, D),
                    W3, b3.reshape(1, D), Wl, bl.reshape(OUT, 1))
    return logits_t.transpose(0, 2, 1).reshape(N, OUT)


# speedup vs baseline: 45.2377x; 1.0153x over previous
import functools

import jax
import jax.numpy as jnp
from jax import lax
from jax.experimental import pallas as pl
from jax.experimental.pallas import tpu as pltpu
from jax.experimental.pallas import tpu_sc as plsc

N = 10000
D = 128
E = 320000
OUT = 2

NC = 2
NS = 16
NW = NC * NS
EPW = E // NW
CH = 128
PAD = 240
NCH = (EPW + PAD) // CH
IH = 40
JUNK = 64
NACC = N + JUNK

SLAB = 624

BLK = 2000
GRID = N // BLK


def _sc_mesh():
    return plsc.VectorSubcoreMesh(core_axis_name="c", subcore_axis_name="s")


DROWS = E // CH
DIB = 26


def _deg_body(edges_hbm, zeros_hbm, out_hbm, idx_v, ones_v, acc_s):
    cid = lax.axis_index("c")
    sid = lax.axis_index("s")
    wid = cid * NS + sid
    for j in range(CH // 16):
        ones_v[pl.ds(j * 16, 16)] = jnp.ones((16,), jnp.float32)

    @pl.when(sid == 0)
    def _():
        pltpu.sync_copy(zeros_hbm, acc_s)

    plsc.subcore_barrier()

    start = 78 * wid + jnp.minimum(wid, 4)
    count = 78 + jnp.where(wid < 4, 1, 0)

    def inner(j, carry):
        pltpu.sync_copy(ones_v, acc_s.at[idx_v.at[j, 1]], add=True)
        return carry

    for r0 in range(0, 79, DIB):
        nrows = jnp.clip(count - r0, 0, DIB)

        @pl.when(nrows > 0)
        def _():
            pltpu.sync_copy(edges_hbm.at[pl.ds(start + r0, DIB)], idx_v)
            lax.fori_loop(0, nrows, inner, 0)

    plsc.subcore_barrier()

    @pl.when(sid == 0)
    def _():
        pltpu.sync_copy(acc_s, out_hbm.at[cid])


_deg_kernel = functools.partial(
    pl.kernel,
    mesh=_sc_mesh(),
    out_type=jax.ShapeDtypeStruct((NC, NACC), jnp.float32),
    scratch_types=[
        pltpu.VMEM((DIB, 2, CH), jnp.int32),
        pltpu.VMEM((CH,), jnp.float32),
        pltpu.VMEM_SHARED((NACC,), jnp.float32),
    ],
)(_deg_body)


def _agg_body(src_hbm, dst_hbm, g_hbm, zeros_hbm, out_hbm,
              sidx_v, didx_v, rows_a, rows_b, acc_s, sem_a, sem_b):
    cid = lax.axis_index("c")
    sid = lax.axis_index("s")
    wid = cid * NS + sid
    tail = N - NS * SLAB

    @pl.when(cid == 0)
    def _():
        pltpu.sync_copy(g_hbm.at[pl.ds(sid * SLAB, SLAB)],
                        acc_s.at[pl.ds(sid * SLAB, SLAB)])

        @pl.when(sid == NS - 1)
        def _():
            pltpu.sync_copy(g_hbm.at[pl.ds(NS * SLAB, tail)],
                            acc_s.at[pl.ds(NS * SLAB, tail)])
            pltpu.sync_copy(zeros_hbm.at[pl.ds(0, JUNK)],
                            acc_s.at[pl.ds(N, JUNK)])

    @pl.when(cid == 1)
    def _():
        pltpu.sync_copy(zeros_hbm,
                        acc_s.at[pl.ds(sid * SLAB, SLAB)])

        @pl.when(sid == NS - 1)
        def _():
            pltpu.sync_copy(zeros_hbm.at[pl.ds(0, tail + JUNK)],
                            acc_s.at[pl.ds(NS * SLAB, tail + JUNK)])

    plsc.subcore_barrier()

    def gather(j, buf, sem):
        pltpu.async_copy(g_hbm.at[sidx_v.at[j]], buf, sem)

    def wait_gather(buf, sem):
        pltpu.make_async_copy(g_hbm.at[sidx_v.at[0]], buf, sem).wait()

    def run_block(nrows):
        gather(0, rows_a, sem_a)
        gather(1, rows_b, sem_b)

        def body(k, carry):
            j0 = 2 * k
            j1 = j0 + 1
            wait_gather(rows_a, sem_a)
            pltpu.sync_copy(rows_a, acc_s.at[didx_v.at[j0]], add=True)

            @pl.when(j0 + 2 < nrows)
            def _():
                gather(j0 + 2, rows_a, sem_a)

            wait_gather(rows_b, sem_b)
            pltpu.sync_copy(rows_b, acc_s.at[didx_v.at[j1]], add=True)

            @pl.when(j1 + 2 < nrows)
            def _():
                gather(j1 + 2, rows_b, sem_b)

            return carry

        lax.fori_loop(0, nrows // 2, body, 0)

    for r0 in range(0, NCH, IH):
        pltpu.sync_copy(src_hbm.at[wid, pl.ds(r0, IH)], sidx_v)
        pltpu.sync_copy(dst_hbm.at[wid, pl.ds(r0, IH)], didx_v)
        run_block(IH)

    plsc.subcore_barrier()

    pltpu.sync_copy(acc_s.at[pl.ds(sid * SLAB, SLAB)],
                    out_hbm.at[cid, pl.ds(sid * SLAB, SLAB)])

    @pl.when(sid == NS - 1)
    def _():
        pltpu.sync_copy(acc_s.at[pl.ds(NS * SLAB, tail)],
                        out_hbm.at[cid, pl.ds(NS * SLAB, tail)])


_agg_kernel = functools.partial(
    pl.kernel,
    mesh=_sc_mesh(),
    out_type=jax.ShapeDtypeStruct((NC, N, D), jnp.float32),
    compiler_params=pltpu.CompilerParams(use_tc_tiling_on_sc=True),
    scratch_types=[
        pltpu.VMEM((IH, CH), jnp.int32),
        pltpu.VMEM((IH, CH), jnp.int32),
        pltpu.VMEM((CH, D), jnp.float32),
        pltpu.VMEM((CH, D), jnp.float32),
        pltpu.VMEM_SHARED((NACC, D), jnp.float32),
        pltpu.SemaphoreType.DMA,
        pltpu.SemaphoreType.DMA,
    ],
)(_agg_body)


def _mmh_body(x_ref, w1_ref, h_ref):
    h_ref[...] = jnp.dot(x_ref[...], w1_ref[...],
                         preferred_element_type=jnp.float32)


def _mmh(x, W1):
    return pl.pallas_call(
        _mmh_body,
        grid=(GRID,),
        in_specs=[
            pl.BlockSpec((BLK, D), lambda i: (i, 0)),
            pl.BlockSpec((D, D), lambda i: (0, 0)),
        ],
        out_specs=pl.BlockSpec((BLK, D), lambda i: (i, 0)),
        out_shape=jax.ShapeDtypeStruct((N, D), jnp.float32),
    )(x, W1)


def _scale_body(h_ref, d0_ref, d1_ref, g_ref, dinv_ref):
    deg = d0_ref[0] + d1_ref[0] + 1.0
    dinv_row = lax.rsqrt(deg)
    dinv_col = jnp.reshape(dinv_row, (BLK, 1))
    g_ref[...] = h_ref[...] * dinv_col
    dinv_ref[0] = dinv_row


def _scale(h, d0, d1):
    return pl.pallas_call(
        _scale_body,
        grid=(GRID,),
        in_specs=[
            pl.BlockSpec((BLK, D), lambda i: (i, 0)),
            pl.BlockSpec((1, 1, BLK), lambda i: (i, 0, 0)),
            pl.BlockSpec((1, 1, BLK), lambda i: (i, 0, 0)),
        ],
        out_specs=[
            pl.BlockSpec((BLK, D), lambda i: (i, 0)),
            pl.BlockSpec((1, 1, BLK), lambda i: (i, 0, 0)),
        ],
        out_shape=[
            jax.ShapeDtypeStruct((N, D), jnp.float32),
            jax.ShapeDtypeStruct((GRID, 1, BLK), jnp.float32),
        ],
    )(h, d0, d1)


def _mlp_body(p0_ref, p1_ref, dinv_ref, b1_ref, w2_ref, b2_ref,
              w3_ref, b3_ref, wl_ref, bl_ref, out_ref):
    dinv_col = jnp.reshape(dinv_ref[0], (BLK, 1))
    conv = (p0_ref[0] + p1_ref[0]) * dinv_col + b1_ref[...]
    h = jnp.maximum(conv, 0.0)
    h = jnp.dot(h, w2_ref[...], preferred_element_type=jnp.float32) + b2_ref[...]
    h = jnp.maximum(h, 0.0)
    h = jnp.dot(h, w3_ref[...], preferred_element_type=jnp.float32) + b3_ref[...]
    h = jnp.maximum(h, 0.0)
    out_ref[0] = (
        lax.dot_general(wl_ref[...], h, (((0,), (1,)), ((), ())),
                        preferred_element_type=jnp.float32)
        + bl_ref[...]
    )


def _mlp(parts, dinv, b1, W2, b2, W3, b3, Wl, bl):
    return pl.pallas_call(
        _mlp_body,
        grid=(GRID,),
        in_specs=[
            pl.BlockSpec((1, BLK, D), lambda i: (0, i, 0)),
            pl.BlockSpec((1, BLK, D), lambda i: (1, i, 0)),
            pl.BlockSpec((1, 1, BLK), lambda i: (i, 0, 0)),
            pl.BlockSpec((1, D), lambda i: (0, 0)),
            pl.BlockSpec((D, D), lambda i: (0, 0)),
            pl.BlockSpec((1, D), lambda i: (0, 0)),
            pl.BlockSpec((D, D), lambda i: (0, 0)),
            pl.BlockSpec((1, D), lambda i: (0, 0)),
            pl.BlockSpec((D, OUT), lambda i: (0, 0)),
            pl.BlockSpec((OUT, 1), lambda i: (0, 0)),
        ],
        out_specs=pl.BlockSpec((1, OUT, BLK), lambda i: (i, 0, 0)),
        out_shape=jax.ShapeDtypeStruct((GRID, OUT, BLK), jnp.float32),
    )(parts, parts, dinv, b1, W2, b2, W3, b3, Wl, bl)


def kernel(x, edge_index, W1, b1, W2, b2, W3, b3, Wl, bl):
    ei = edge_index.astype(jnp.int32)
    padi = jnp.arange(PAD, dtype=jnp.int32) % JUNK
    src_pad = jnp.broadcast_to(padi, (NW, PAD))
    dst_pad = jnp.broadcast_to(N + padi, (NW, PAD))
    src3 = jnp.concatenate(
        [ei[0].reshape(NW, EPW), src_pad], axis=1).reshape(NW, NCH, CH)
    dst3 = jnp.concatenate(
        [ei[1].reshape(NW, EPW), dst_pad], axis=1).reshape(NW, NCH, CH)
    zeros1 = jnp.zeros((NACC,), jnp.float32)
    zeros2 = jnp.zeros((SLAB, D), jnp.float32)

    eidx3 = ei.reshape(2, DROWS, CH).transpose(1, 0, 2)

    h = _mmh(x, W1)
    degp = _deg_kernel(eidx3, zeros1)
    d0 = degp[0, :N].reshape(GRID, 1, BLK)
    d1 = degp[1, :N].reshape(GRID, 1, BLK)

    g, dinv = _scale(h, d0, d1)

    parts = _agg_kernel(src3, dst3, g, zeros2)

    logits_t = _mlp(parts, dinv,
                    b1.reshape(1, D), W2, b2.reshape(1, D),
                    W3, b3.reshape(1, D), Wl, bl.reshape(OUT, 1))
    return logits_t.transpose(0, 2, 1).reshape(N, OUT)


# speedup vs baseline: 45.2598x; 1.0005x over previous
import functools

import jax
import jax.numpy as jnp
from jax import lax
from jax.experimental import pallas as pl
from jax.experimental.pallas import tpu as pltpu
from jax.experimental.pallas import tpu_sc as plsc

N = 10000
D = 128
E = 320000
OUT = 2

NC = 2
NS = 16
NW = NC * NS
EPW = E // NW
CH = 128
PAD = 240
NCH = (EPW + PAD) // CH
IH = 64
JUNK = 64
NACC = N + JUNK

SLAB = 624

BLK = 2000
GRID = N // BLK


def _sc_mesh():
    return plsc.VectorSubcoreMesh(core_axis_name="c", subcore_axis_name="s")


DROWS = E // CH
DIB = 26


def _deg_body(edges_hbm, zeros_hbm, out_hbm, idx_v, ones_v, acc_s):
    cid = lax.axis_index("c")
    sid = lax.axis_index("s")
    wid = cid * NS + sid
    for j in range(CH // 16):
        ones_v[pl.ds(j * 16, 16)] = jnp.ones((16,), jnp.float32)

    @pl.when(sid == 0)
    def _():
        pltpu.sync_copy(zeros_hbm, acc_s)

    plsc.subcore_barrier()

    start = 78 * wid + jnp.minimum(wid, 4)
    count = 78 + jnp.where(wid < 4, 1, 0)

    def inner(j, carry):
        pltpu.sync_copy(ones_v, acc_s.at[idx_v.at[j, 1]], add=True)
        return carry

    for r0 in range(0, 79, DIB):
        nrows = jnp.clip(count - r0, 0, DIB)

        @pl.when(nrows > 0)
        def _():
            pltpu.sync_copy(edges_hbm.at[pl.ds(start + r0, DIB)], idx_v)
            lax.fori_loop(0, nrows, inner, 0)

    plsc.subcore_barrier()

    @pl.when(sid == 0)
    def _():
        pltpu.sync_copy(acc_s, out_hbm.at[cid])


_deg_kernel = functools.partial(
    pl.kernel,
    mesh=_sc_mesh(),
    out_type=jax.ShapeDtypeStruct((NC, NACC), jnp.float32),
    scratch_types=[
        pltpu.VMEM((DIB, 2, CH), jnp.int32),
        pltpu.VMEM((CH,), jnp.float32),
        pltpu.VMEM_SHARED((NACC,), jnp.float32),
    ],
)(_deg_body)


def _agg_body(src_hbm, dst_hbm, g_hbm, zeros_hbm, out_hbm,
              sidx_v, didx_v, rows_a, rows_b, acc_s, sem_a, sem_b):
    cid = lax.axis_index("c")
    sid = lax.axis_index("s")
    wid = cid * NS + sid
    tail = N - NS * SLAB

    @pl.when(cid == 0)
    def _():
        pltpu.sync_copy(g_hbm.at[pl.ds(sid * SLAB, SLAB)],
                        acc_s.at[pl.ds(sid * SLAB, SLAB)])

        @pl.when(sid == NS - 1)
        def _():
            pltpu.sync_copy(g_hbm.at[pl.ds(NS * SLAB, tail)],
                            acc_s.at[pl.ds(NS * SLAB, tail)])
            pltpu.sync_copy(zeros_hbm.at[pl.ds(0, JUNK)],
                            acc_s.at[pl.ds(N, JUNK)])

    @pl.when(cid == 1)
    def _():
        pltpu.sync_copy(zeros_hbm,
                        acc_s.at[pl.ds(sid * SLAB, SLAB)])

        @pl.when(sid == NS - 1)
        def _():
            pltpu.sync_copy(zeros_hbm.at[pl.ds(0, tail + JUNK)],
                            acc_s.at[pl.ds(NS * SLAB, tail + JUNK)])

    plsc.subcore_barrier()

    def gather(j, buf, sem):
        pltpu.async_copy(g_hbm.at[sidx_v.at[j]], buf, sem)

    def wait_gather(buf, sem):
        pltpu.make_async_copy(g_hbm.at[sidx_v.at[0]], buf, sem).wait()

    def run_block(nrows):
        gather(0, rows_a, sem_a)
        gather(1, rows_b, sem_b)

        def body(k, carry):
            j0 = 2 * k
            j1 = j0 + 1
            wait_gather(rows_a, sem_a)
            pltpu.sync_copy(rows_a, acc_s.at[didx_v.at[j0]], add=True)

            @pl.when(j0 + 2 < nrows)
            def _():
                gather(j0 + 2, rows_a, sem_a)

            wait_gather(rows_b, sem_b)
            pltpu.sync_copy(rows_b, acc_s.at[didx_v.at[j1]], add=True)

            @pl.when(j1 + 2 < nrows)
            def _():
                gather(j1 + 2, rows_b, sem_b)

            return carry

        lax.fori_loop(0, nrows // 2, body, 0)

    for r0, nrows in ((0, IH), (IH, NCH - IH)):
        pltpu.sync_copy(src_hbm.at[wid, pl.ds(r0, nrows)],
                        sidx_v.at[pl.ds(0, nrows)])
        pltpu.sync_copy(dst_hbm.at[wid, pl.ds(r0, nrows)],
                        didx_v.at[pl.ds(0, nrows)])
        run_block(nrows)

    plsc.subcore_barrier()

    pltpu.sync_copy(acc_s.at[pl.ds(sid * SLAB, SLAB)],
                    out_hbm.at[cid, pl.ds(sid * SLAB, SLAB)])

    @pl.when(sid == NS - 1)
    def _():
        pltpu.sync_copy(acc_s.at[pl.ds(NS * SLAB, tail)],
                        out_hbm.at[cid, pl.ds(NS * SLAB, tail)])


_agg_kernel = functools.partial(
    pl.kernel,
    mesh=_sc_mesh(),
    out_type=jax.ShapeDtypeStruct((NC, N, D), jnp.float32),
    compiler_params=pltpu.CompilerParams(use_tc_tiling_on_sc=True),
    scratch_types=[
        pltpu.VMEM((IH, CH), jnp.int32),
        pltpu.VMEM((IH, CH), jnp.int32),
        pltpu.VMEM((CH, D), jnp.float32),
        pltpu.VMEM((CH, D), jnp.float32),
        pltpu.VMEM_SHARED((NACC, D), jnp.float32),
        pltpu.SemaphoreType.DMA,
        pltpu.SemaphoreType.DMA,
    ],
)(_agg_body)


def _mmh_body(x_ref, w1_ref, h_ref):
    h_ref[...] = jnp.dot(x_ref[...], w1_ref[...],
                         preferred_element_type=jnp.float32)


def _mmh(x, W1):
    return pl.pallas_call(
        _mmh_body,
        grid=(GRID,),
        in_specs=[
            pl.BlockSpec((BLK, D), lambda i: (i, 0)),
            pl.BlockSpec((D, D), lambda i: (0, 0)),
        ],
        out_specs=pl.BlockSpec((BLK, D), lambda i: (i, 0)),
        out_shape=jax.ShapeDtypeStruct((N, D), jnp.float32),
    )(x, W1)


def _scale_body(h_ref, d0_ref, d1_ref, g_ref, dinv_ref):
    deg = d0_ref[0] + d1_ref[0] + 1.0
    dinv_row = lax.rsqrt(deg)
    dinv_col = jnp.reshape(dinv_row, (BLK, 1))
    g_ref[...] = h_ref[...] * dinv_col
    dinv_ref[0] = dinv_row


def _scale(h, d0, d1):
    return pl.pallas_call(
        _scale_body,
        grid=(GRID,),
        in_specs=[
            pl.BlockSpec((BLK, D), lambda i: (i, 0)),
            pl.BlockSpec((1, 1, BLK), lambda i: (i, 0, 0)),
            pl.BlockSpec((1, 1, BLK), lambda i: (i, 0, 0)),
        ],
        out_specs=[
            pl.BlockSpec((BLK, D), lambda i: (i, 0)),
            pl.BlockSpec((1, 1, BLK), lambda i: (i, 0, 0)),
        ],
        out_shape=[
            jax.ShapeDtypeStruct((N, D), jnp.float32),
            jax.ShapeDtypeStruct((GRID, 1, BLK), jnp.float32),
        ],
    )(h, d0, d1)


def _mlp_body(p0_ref, p1_ref, dinv_ref, b1_ref, w2_ref, b2_ref,
              w3_ref, b3_ref, wl_ref, bl_ref, out_ref):
    dinv_col = jnp.reshape(dinv_ref[0], (BLK, 1))
    conv = (p0_ref[0] + p1_ref[0]) * dinv_col + b1_ref[...]
    h = jnp.maximum(conv, 0.0)
    h = jnp.dot(h, w2_ref[...], preferred_element_type=jnp.float32) + b2_ref[...]
    h = jnp.maximum(h, 0.0)
    h = jnp.dot(h, w3_ref[...], preferred_element_type=jnp.float32) + b3_ref[...]
    h = jnp.maximum(h, 0.0)
    out_ref[0] = (
        lax.dot_general(wl_ref[...], h, (((0,), (1,)), ((), ())),
                        preferred_element_type=jnp.float32)
        + bl_ref[...]
    )


def _mlp(parts, dinv, b1, W2, b2, W3, b3, Wl, bl):
    return pl.pallas_call(
        _mlp_body,
        grid=(GRID,),
        in_specs=[
            pl.BlockSpec((1, BLK, D), lambda i: (0, i, 0)),
            pl.BlockSpec((1, BLK, D), lambda i: (1, i, 0)),
            pl.BlockSpec((1, 1, BLK), lambda i: (i, 0, 0)),
            pl.BlockSpec((1, D), lambda i: (0, 0)),
            pl.BlockSpec((D, D), lambda i: (0, 0)),
            pl.BlockSpec((1, D), lambda i: (0, 0)),
            pl.BlockSpec((D, D), lambda i: (0, 0)),
            pl.BlockSpec((1, D), lambda i: (0, 0)),
            pl.BlockSpec((D, OUT), lambda i: (0, 0)),
            pl.BlockSpec((OUT, 1), lambda i: (0, 0)),
        ],
        out_specs=pl.BlockSpec((1, OUT, BLK), lambda i: (i, 0, 0)),
        out_shape=jax.ShapeDtypeStruct((GRID, OUT, BLK), jnp.float32),
    )(parts, parts, dinv, b1, W2, b2, W3, b3, Wl, bl)


def kernel(x, edge_index, W1, b1, W2, b2, W3, b3, Wl, bl):
    ei = edge_index.astype(jnp.int32)
    padi = jnp.arange(PAD, dtype=jnp.int32) % JUNK
    src_pad = jnp.broadcast_to(padi, (NW, PAD))
    dst_pad = jnp.broadcast_to(N + padi, (NW, PAD))
    src3 = jnp.concatenate(
        [ei[0].reshape(NW, EPW), src_pad], axis=1).reshape(NW, NCH, CH)
    dst3 = jnp.concatenate(
        [ei[1].reshape(NW, EPW), dst_pad], axis=1).reshape(NW, NCH, CH)
    zeros1 = jnp.zeros((NACC,), jnp.float32)
    zeros2 = jnp.zeros((SLAB, D), jnp.float32)

    eidx3 = ei.reshape(2, DROWS, CH).transpose(1, 0, 2)

    h = _mmh(x, W1)
    degp = _deg_kernel(eidx3, zeros1)
    d0 = degp[0, :N].reshape(GRID, 1, BLK)
    d1 = degp[1, :N].reshape(GRID, 1, BLK)

    g, dinv = _scale(h, d0, d1)

    parts = _agg_kernel(src3, dst3, g, zeros2)

    logits_t = _mlp(parts, dinv,
                    b1.reshape(1, D), W2, b2.reshape(1, D),
                    W3, b3.reshape(1, D), Wl, bl.reshape(OUT, 1))
    return logits_t.transpose(0, 2, 1).reshape(N, OUT)
